# Initial kernel scaffold; baseline (speedup 1.0000x reference)
import jax
import jax.numpy as jnp
from jax.experimental import pallas as pl


def kernel(x, edge_index, matrix, batch, params):
    raise NotImplementedError("write your pallas kernel here")


# speedup vs baseline: 1.5735x; 1.5735x over previous
import functools

import jax
import jax.numpy as jnp
from jax import lax
from jax.experimental import pallas as pl
from jax.experimental.pallas import tpu as pltpu
from jax.experimental.pallas import tpu_sc as plsc

N = 10000
NP = 10240
E = 50000
EP = 50176
TPW = 1568
NCH = 14
CH = 112
BPT = NP // 32
EPL = EP + CH
NB = 16
GW = 128
F32 = jnp.float32


def _leaky_silu(v, alpha):
    return v * jax.nn.sigmoid(v) + alpha * v


@functools.lru_cache(maxsize=None)
def _make_gather():
    mesh = plsc.VectorSubcoreMesh(
        core_axis_name="c", subcore_axis_name="s", num_cores=2)

    @functools.partial(
        pl.kernel,
        out_type=jax.ShapeDtypeStruct((EP, GW), F32),
        mesh=mesh,
        scratch_types=[
            pltpu.VMEM((NCH, CH), jnp.int32),
            pltpu.VMEM((CH, GW), F32),
            pltpu.VMEM((CH, GW), F32),
            pltpu.SemaphoreType.DMA,
            pltpu.SemaphoreType.DMA,
        ],
    )
    def gk(table_hbm, idx_hbm, out_hbm, idx_v, buf0, buf1, sem0, sem1):
        c = lax.axis_index("c")
        s = lax.axis_index("s")
        base = (c * 16 + s) * TPW
        pltpu.sync_copy(idx_hbm.at[c, s], idx_v)
        bufs = (buf0, buf1)
        sems = (sem0, sem1)
        cps = [None, None]
        for j in range(NCH):
            k = j % 2
            if cps[k] is not None:
                cps[k].wait()
                pltpu.sync_copy(bufs[k], out_hbm.at[pl.ds(base + (j - 2) * CH, CH)])
            cps[k] = pltpu.async_copy(
                table_hbm.at[idx_v.at[j]], bufs[k], sems[k])
        for j in range(NCH - 2, NCH):
            k = j % 2
            cps[k].wait()
            pltpu.sync_copy(bufs[k], out_hbm.at[pl.ds(base + j * CH, CH)])

    return gk


@functools.lru_cache(maxsize=None)
def _make_bin():
    mesh = plsc.VectorSubcoreMesh(
        core_axis_name="c", subcore_axis_name="s", num_cores=2)

    @functools.partial(
        pl.kernel,
        out_type=(jax.ShapeDtypeStruct((32 * EPL,), jnp.int32),
                  jax.ShapeDtypeStruct((32 * 16,), jnp.int32)),
        mesh=mesh,
        compiler_params=pltpu.CompilerParams(needs_layout_passes=False),
        scratch_types=[
            pltpu.VMEM((TPW,), jnp.int32),
            pltpu.VMEM((EPL,), jnp.int32),
            pltpu.VMEM((16,), jnp.int32),
        ],
    )
    def bk(dst_hbm, lists_hbm, cnts_hbm, idx_v, list_v, cnt_v):
        c = lax.axis_index("c")
        s = lax.axis_index("s")
        w = c * 16 + s
        rbase = w * BPT
        cnt = jnp.int32(0)
        for ch in range(EP // TPW):
            pltpu.sync_copy(dst_hbm.at[pl.ds(ch * TPW, TPW)], idx_v)

            def scan_g(g, cnt):
                v = idx_v[pl.ds(g * 16, 16)]
                lv = v - rbase
                m = (lv >= 0) & (lv < BPT)
                eid = (lax.broadcasted_iota(jnp.int32, (16,), 0)
                       + (ch * TPW + g * 16))
                packed = eid * 512 + jnp.where(m, lv, 0)
                mc = jnp.cumsum(m.astype(jnp.int32))
                plsc.store_scatter(list_v, [cnt + mc - 1], packed, mask=m)
                return cnt + mc[15]

            cnt = lax.fori_loop(0, TPW // 16, scan_g, cnt)
        dummy = jnp.full((16,), BPT, jnp.int32)
        for e in range(CH // 16):
            list_v[pl.ds(cnt + e * 16, 16)] = dummy
        cnt_v[...] = jnp.full((16,), cnt, jnp.int32)
        pltpu.sync_copy(list_v, lists_hbm.at[pl.ds(w * EPL, EPL)])
        pltpu.sync_copy(cnt_v, cnts_hbm.at[pl.ds(w * 16, 16)])

    return bk


@functools.lru_cache(maxsize=None)
def _make_scatter():
    mesh = plsc.VectorSubcoreMesh(
        core_axis_name="c", subcore_axis_name="s", num_cores=2)

    @functools.partial(
        pl.kernel,
        out_type=jax.ShapeDtypeStruct((NP, GW), F32),
        mesh=mesh,
        compiler_params=pltpu.CompilerParams(needs_layout_passes=False),
        scratch_types=[
            pltpu.VMEM((16,), jnp.int32),
            pltpu.VMEM((CH,), jnp.int32),
            pltpu.VMEM((CH,), jnp.int32),
            pltpu.VMEM((CH,), jnp.int32),
            pltpu.VMEM((CH, GW), F32),
            pltpu.VMEM((BPT + 8, GW), F32),
            pltpu.SemaphoreType.DMA,
        ],
    )
    def sk(msg_hbm, lists_hbm, cnts_hbm, out_hbm, cnt_v, pk_v, eid_v, lv_v,
           msg_v, acc, sem):
        c = lax.axis_index("c")
        s = lax.axis_index("s")
        w = c * 16 + s

        def zero_row(i, carry):
            for k in range(GW // 16):
                acc[i, pl.ds(k * 16, 16)] = jnp.zeros((16,), F32)
            return carry

        lax.fori_loop(0, BPT + 8, zero_row, 0)

        pltpu.sync_copy(cnts_hbm.at[pl.ds(w * 16, 16)], cnt_v)
        cnt = cnt_v[...][0]
        nch = lax.div(cnt + (CH - 1), jnp.int32(CH))

        def chunk(j, carry):
            pltpu.sync_copy(lists_hbm.at[pl.ds(w * EPL + j * CH, CH)], pk_v)

            def unpack(g, carry2):
                pk = pk_v[pl.ds(g * 16, 16)]
                eid_v[pl.ds(g * 16, 16)] = lax.shift_right_logical(pk, 9)
                lv_v[pl.ds(g * 16, 16)] = lax.bitwise_and(
                    pk, jnp.full((16,), 511, jnp.int32))
                return carry2

            lax.fori_loop(0, CH // 16, unpack, 0)
            pltpu.async_copy(msg_hbm.at[eid_v], msg_v, sem).wait()

            def edge_group(g, carry2):
                lv16 = lv_v[pl.ds(g * 16, 16)]
                for e2 in range(16):
                    lv = lv16[e2]
                    row = g * 16 + e2
                    for k in range(GW // 16):
                        acc[lv, pl.ds(k * 16, 16)] += (
                            msg_v[row, pl.ds(k * 16, 16)])
                return carry2

            lax.fori_loop(0, CH // 16, edge_group, 0)
            return carry

        lax.fori_loop(0, nch, chunk, 0)
        pltpu.sync_copy(acc.at[pl.ds(0, BPT)],
                        out_hbm.at[pl.ds(w * BPT, BPT)])

    return sk


def _dot(a, b, dims, precision=None):
    return lax.dot_general(a, b, (dims, ((), ())),
                           preferred_element_type=F32, precision=precision)


def _prep_call(x_p, batch_2d, mat_flat):
    BN = NP // 4

    def body(xb, bb, mat, out):
        xv = xb[...]
        bcol = bb[...]
        oh = (lax.broadcasted_iota(jnp.int32, (BN, NB), 1)
              == jnp.broadcast_to(bcol, (BN, NB))).astype(F32)
        pos = _dot(oh, mat[...], ((1,), (0,)),
                   precision=lax.Precision.HIGHEST)
        cols = [xv[:, 0:1]]
        for k in range(3):
            col = (xv[:, 1:2] * pos[:, k:k + 1]
                   + xv[:, 2:3] * pos[:, 3 + k:4 + k]
                   + xv[:, 3:4] * pos[:, 6 + k:7 + k])
            cols.append(col)
        cols.append(jnp.zeros((BN, GW - 4), F32))
        out[...] = jnp.concatenate(cols, axis=1)

    return pl.pallas_call(
        body,
        grid=(4,),
        in_specs=[
            pl.BlockSpec((BN, 4), lambda i: (i, 0)),
            pl.BlockSpec((BN, 1), lambda i: (i, 0)),
            pl.BlockSpec((NB, 9), lambda i: (0, 0)),
        ],
        out_specs=pl.BlockSpec((BN, GW), lambda i: (i, 0)),
        out_shape=jax.ShapeDtypeStruct((NP, GW), F32),
    )(x_p, batch_2d, mat_flat)


@functools.lru_cache(maxsize=None)
def _make_conv(ic, oc, BE):
    F = ic * oc
    grid = EP // BE

    def body(hs, hd, hsi, Wa, ba, Wb, bb, out):
        ew = (hd[...] - hs[...])[:, 1:4]
        e1 = _leaky_silu(_dot(ew, Wa[...], ((1,), (1,))) + ba[...], 0.05)
        e2 = _leaky_silu(_dot(e1, Wb[...], ((1,), (1,))) + bb[...], 0.05)
        h = hsi[...]
        acc = h[:, 0:1] * e2[:, 0:oc]
        for i in range(1, ic):
            acc = acc + h[:, i:i + 1] * e2[:, i * oc:(i + 1) * oc]
        if GW > oc:
            acc = jnp.concatenate(
                [acc, jnp.zeros((BE, GW - oc), F32)], axis=1)
        out[...] = acc

    return pl.pallas_call(
        body,
        grid=(grid,),
        in_specs=[
            pl.BlockSpec((BE, GW), lambda i: (i, 0)),
            pl.BlockSpec((BE, GW), lambda i: (i, 0)),
            pl.BlockSpec((BE, GW), lambda i: (i, 0)),
            pl.BlockSpec((64, 3), lambda i: (0, 0)),
            pl.BlockSpec((1, 64), lambda i: (0, 0)),
            pl.BlockSpec((F, 64), lambda i: (0, 0)),
            pl.BlockSpec((1, F), lambda i: (0, 0)),
        ],
        out_specs=pl.BlockSpec((BE, GW), lambda i: (i, 0)),
        out_shape=jax.ShapeDtypeStruct((EP, GW), F32),
    )


@functools.lru_cache(maxsize=None)
def _make_node(ic, oc):
    BN = NP // 4

    def body(agg, hin, root, bias, W1, b1, W2, b2, out):
        asum = agg[...][:, :oc]
        h = hin[...][:, :ic]
        z = _leaky_silu(asum + _dot(h, root[...], ((1,), (1,)))
                        + bias[...], 0.1)
        val = _leaky_silu(_dot(_leaky_silu(_dot(z, W1[...], ((1,), (1,)))
                                           + b1[...], 0.05),
                               W2[...], ((1,), (1,))) + b2[...], 0.05) + z
        if GW > oc:
            val = jnp.concatenate(
                [val, jnp.zeros((BN, GW - oc), F32)], axis=1)
        out[...] = val

    return pl.pallas_call(
        body,
        grid=(4,),
        in_specs=[
            pl.BlockSpec((BN, GW), lambda i: (i, 0)),
            pl.BlockSpec((BN, GW), lambda i: (i, 0)),
            pl.BlockSpec((oc, ic), lambda i: (0, 0)),
            pl.BlockSpec((1, oc), lambda i: (0, 0)),
            pl.BlockSpec((128, oc), lambda i: (0, 0)),
            pl.BlockSpec((1, 128), lambda i: (0, 0)),
            pl.BlockSpec((oc, 128), lambda i: (0, 0)),
            pl.BlockSpec((1, oc), lambda i: (0, 0)),
        ],
        out_specs=pl.BlockSpec((BN, GW), lambda i: (i, 0)),
        out_shape=jax.ShapeDtypeStruct((NP, GW), F32),
    )


def _readout_call(h3, batch_2d, fcW, fcb):
    BN = NP // 4

    def body(h, bb, W, b, out, g):
        i = pl.program_id(0)

        @pl.when(i == 0)
        def _():
            g[...] = jnp.zeros((NB, 128), F32)

        bcol = bb[...]
        oh = (lax.broadcasted_iota(jnp.int32, (BN, NB), 1)
              == jnp.broadcast_to(bcol, (BN, NB))).astype(F32)
        g[...] += _dot(oh, h[...], ((0,), (0,)),
                       precision=lax.Precision.HIGHEST)
        val = (jnp.sum(g[...] * W[...], axis=1, keepdims=True)
               + jnp.broadcast_to(b[...], (NB, 1)))
        out[...] = -_leaky_silu(val, 0.1)

    return pl.pallas_call(
        body,
        grid=(4,),
        in_specs=[
            pl.BlockSpec((BN, GW), lambda i: (i, 0)),
            pl.BlockSpec((BN, 1), lambda i: (i, 0)),
            pl.BlockSpec((1, 128), lambda i: (0, 0)),
            pl.BlockSpec((1, 1), lambda i: (0, 0)),
        ],
        out_specs=pl.BlockSpec((NB, 1), lambda i: (0, 0)),
        out_shape=jax.ShapeDtypeStruct((NB, 1), F32),
        scratch_shapes=[pltpu.VMEM((NB, 128), F32)],
    )(h3, batch_2d, fcW, fcb)


def kernel(x, edge_index, matrix, batch, params):
    p = params
    i32 = jnp.int32
    src = edge_index[0]
    dst = edge_index[1]
    src_r = jnp.concatenate(
        [src, jnp.zeros((EP - E,), i32)]).reshape(2, 16, NCH, CH)
    dst_p = jnp.concatenate([dst, jnp.full((EP - E,), N, i32)])
    dst_r = dst_p.reshape(2, 16, NCH, CH)
    x_p = jnp.concatenate([x, jnp.zeros((NP - N, 4), F32)], axis=0)
    batch_2d = jnp.concatenate(
        [batch, jnp.full((NP - N,), NB, i32)]).reshape(NP, 1)
    mat_flat = matrix.reshape(NB, 9)

    h0 = _prep_call(x_p, batch_2d, mat_flat)

    gather = _make_gather()
    lists, cnts = _make_bin()(dst_p)
    scatter_b = _make_scatter()

    def scatter(msg, _):
        return scatter_b(msg, lists, cnts)

    hs = gather(h0, src_r)
    hd = gather(h0, dst_r)

    def r2(v):
        return v.reshape(1, -1)

    msg1 = _make_conv(4, 8, 1024)(hs, hd, hs, p['c1_Wa'], r2(p['c1_ba']),
                                  p['c1_Wb'], r2(p['c1_bb']))
    agg1 = scatter(msg1, dst_r)
    h1 = _make_node(4, 8)(
        agg1, h0, p['c1_root'], r2(p['c1_bias']),
        p['il1_W1'], r2(p['il1_b1']), p['il1_W2'], r2(p['il1_b2']))

    hs8 = gather(h1, src_r)
    msg2 = _make_conv(8, 64, 1024)(hs, hd, hs8, p['c2_Wa'], r2(p['c2_ba']),
                                   p['c2_Wb'], r2(p['c2_bb']))
    agg2 = scatter(msg2, dst_r)
    h2 = _make_node(8, 64)(
        agg2, h1, p['c2_root'], r2(p['c2_bias']),
        p['il2_W1'], r2(p['il2_b1']), p['il2_W2'], r2(p['il2_b2']))

    hs64 = gather(h2, src_r)
    msg3 = _make_conv(64, 128, 256)(hs, hd, hs64, p['c3_Wa'], r2(p['c3_ba']),
                                    p['c3_Wb'], r2(p['c3_bb']))
    agg3 = scatter(msg3, dst_r)
    h3 = _make_node(64, 128)(
        agg3, h2, p['c3_root'], r2(p['c3_bias']),
        p['il3_W1'], r2(p['il3_b1']), p['il3_W2'], r2(p['il3_b2']))

    return _readout_call(h3, batch_2d, r2(p['fc1_W']), r2(p['fc1_b']))


# speedup vs baseline: 1.6675x; 1.0598x over previous
import functools

import jax
import jax.numpy as jnp
from jax import lax
from jax.experimental import pallas as pl
from jax.experimental.pallas import tpu as pltpu
from jax.experimental.pallas import tpu_sc as plsc

N = 10000
NP = 10240
E = 50000
EP = 50176
TPW = 1568
NCH = 14
CH = 112
BPT = NP // 32
EPL = EP + CH
NB = 16
GW = 128
F32 = jnp.float32


def _leaky_silu(v, alpha):
    return v * jax.nn.sigmoid(v) + alpha * v


@functools.lru_cache(maxsize=None)
def _make_gather():
    mesh = plsc.VectorSubcoreMesh(
        core_axis_name="c", subcore_axis_name="s", num_cores=2)

    @functools.partial(
        pl.kernel,
        out_type=jax.ShapeDtypeStruct((EP, GW), F32),
        mesh=mesh,
        scratch_types=[
            pltpu.VMEM((NCH, CH), jnp.int32),
            pltpu.VMEM((CH, GW), F32),
            pltpu.VMEM((CH, GW), F32),
            pltpu.SemaphoreType.DMA,
            pltpu.SemaphoreType.DMA,
        ],
    )
    def gk(table_hbm, idx_hbm, out_hbm, idx_v, buf0, buf1, sem0, sem1):
        c = lax.axis_index("c")
        s = lax.axis_index("s")
        base = (c * 16 + s) * TPW
        pltpu.sync_copy(idx_hbm.at[c, s], idx_v)
        bufs = (buf0, buf1)
        sems = (sem0, sem1)
        cps = [None, None]
        for j in range(NCH):
            k = j % 2
            if cps[k] is not None:
                cps[k].wait()
                pltpu.sync_copy(bufs[k], out_hbm.at[pl.ds(base + (j - 2) * CH, CH)])
            cps[k] = pltpu.async_copy(
                table_hbm.at[idx_v.at[j]], bufs[k], sems[k])
        for j in range(NCH - 2, NCH):
            k = j % 2
            cps[k].wait()
            pltpu.sync_copy(bufs[k], out_hbm.at[pl.ds(base + j * CH, CH)])

    return gk


@functools.lru_cache(maxsize=None)
def _make_bin():
    mesh = plsc.VectorSubcoreMesh(
        core_axis_name="c", subcore_axis_name="s", num_cores=2)

    @functools.partial(
        pl.kernel,
        out_type=(jax.ShapeDtypeStruct((32 * EPL,), jnp.int32),
                  jax.ShapeDtypeStruct((32 * 16,), jnp.int32)),
        mesh=mesh,
        compiler_params=pltpu.CompilerParams(needs_layout_passes=False),
        scratch_types=[
            pltpu.VMEM((TPW,), jnp.int32),
            pltpu.VMEM((EPL,), jnp.int32),
            pltpu.VMEM((16,), jnp.int32),
        ],
    )
    def bk(dst_hbm, lists_hbm, cnts_hbm, idx_v, list_v, cnt_v):
        c = lax.axis_index("c")
        s = lax.axis_index("s")
        w = c * 16 + s
        rbase = w * BPT
        cnt = jnp.int32(0)
        for ch in range(EP // TPW):
            pltpu.sync_copy(dst_hbm.at[pl.ds(ch * TPW, TPW)], idx_v)

            def scan_g(g, cnt):
                v = idx_v[pl.ds(g * 16, 16)]
                lv = v - rbase
                m = (lv >= 0) & (lv < BPT)
                eid = (lax.broadcasted_iota(jnp.int32, (16,), 0)
                       + (ch * TPW + g * 16))
                packed = eid * 512 + jnp.where(m, lv, 0)
                mc = jnp.cumsum(m.astype(jnp.int32))
                plsc.store_scatter(list_v, [cnt + mc - 1], packed, mask=m)
                return cnt + mc[15]

            cnt = lax.fori_loop(0, TPW // 16, scan_g, cnt)
        dummy = jnp.full((16,), BPT, jnp.int32)
        for e in range(CH // 16):
            list_v[pl.ds(cnt + e * 16, 16)] = dummy
        cnt_v[...] = jnp.full((16,), cnt, jnp.int32)
        pltpu.sync_copy(list_v, lists_hbm.at[pl.ds(w * EPL, EPL)])
        pltpu.sync_copy(cnt_v, cnts_hbm.at[pl.ds(w * 16, 16)])

    return bk


@functools.lru_cache(maxsize=None)
def _make_scatter():
    mesh = plsc.VectorSubcoreMesh(
        core_axis_name="c", subcore_axis_name="s", num_cores=2)

    @functools.partial(
        pl.kernel,
        out_type=jax.ShapeDtypeStruct((NP, GW), F32),
        mesh=mesh,
        compiler_params=pltpu.CompilerParams(needs_layout_passes=False),
        scratch_types=[
            pltpu.VMEM((16,), jnp.int32),
            pltpu.VMEM((CH,), jnp.int32),
            pltpu.VMEM((CH,), jnp.int32),
            pltpu.VMEM((CH,), jnp.int32),
            pltpu.VMEM((CH,), jnp.int32),
            pltpu.VMEM((CH,), jnp.int32),
            pltpu.VMEM((CH, GW), F32),
            pltpu.VMEM((CH, GW), F32),
            pltpu.VMEM((BPT + 8, GW), F32),
            pltpu.SemaphoreType.DMA,
            pltpu.SemaphoreType.DMA,
        ],
    )
    def sk(msg_hbm, lists_hbm, cnts_hbm, out_hbm, cnt_v, pk_v, eid_v, lv_v,
           eid_v2, lv_v2, msg_v, msg_v2, acc, sem, sem2):
        c = lax.axis_index("c")
        s = lax.axis_index("s")
        w = c * 16 + s

        def zero_row(i, carry):
            for k in range(GW // 16):
                acc[i, pl.ds(k * 16, 16)] = jnp.zeros((16,), F32)
            return carry

        lax.fori_loop(0, BPT + 8, zero_row, 0)

        pltpu.sync_copy(cnts_hbm.at[pl.ds(w * 16, 16)], cnt_v)
        cnt = cnt_v[...][0]
        nch = lax.div(cnt + (CH - 1), jnp.int32(CH))

        eid_b = (eid_v, eid_v2)
        lv_b = (lv_v, lv_v2)
        msg_b = (msg_v, msg_v2)
        sem_b = (sem, sem2)

        def load_unpack(jj, b):
            pltpu.sync_copy(lists_hbm.at[pl.ds(w * EPL + jj * CH, CH)], pk_v)

            def unpack(g, carry2):
                pk = pk_v[pl.ds(g * 16, 16)]
                eid_b[b][pl.ds(g * 16, 16)] = lax.shift_right_logical(pk, 9)
                lv_b[b][pl.ds(g * 16, 16)] = lax.bitwise_and(
                    pk, jnp.full((16,), 511, jnp.int32))
                return carry2

            lax.fori_loop(0, CH // 16, unpack, 0)

        def accumulate(b):
            def edge_group(g, carry2):
                lv16 = lv_b[b][pl.ds(g * 16, 16)]
                for e2 in range(16):
                    lv = lv16[e2]
                    row = g * 16 + e2
                    for k in range(GW // 16):
                        plsc.addupdate(acc.at[lv, pl.ds(k * 16, 16)],
                                       msg_b[b][row, pl.ds(k * 16, 16)])
                return carry2

            lax.fori_loop(0, CH // 16, edge_group, 0)

        @pl.when(nch > 0)
        def _():
            load_unpack(jnp.int32(0), 0)
            pltpu.async_copy(msg_hbm.at[eid_b[0]], msg_b[0], sem_b[0])

        def pair(i, carry):
            for b in range(2):
                jj = i * 2 + b
                nb = 1 - b

                @pl.when(jj < nch)
                def _():
                    @pl.when(jj + 1 < nch)
                    def _():
                        load_unpack(jj + 1, nb)
                        pltpu.async_copy(
                            msg_hbm.at[eid_b[nb]], msg_b[nb], sem_b[nb])

                    pltpu.make_async_copy(
                        msg_hbm.at[eid_b[b]], msg_b[b], sem_b[b]).wait()
                    accumulate(b)

            return carry

        lax.fori_loop(0, lax.div(nch + 1, jnp.int32(2)), pair, 0)
        pltpu.sync_copy(acc.at[pl.ds(0, BPT)],
                        out_hbm.at[pl.ds(w * BPT, BPT)])

    return sk


def _dot(a, b, dims, precision=None):
    return lax.dot_general(a, b, (dims, ((), ())),
                           preferred_element_type=F32, precision=precision)


def _prep_call(x_p, batch_2d, mat_flat):
    BN = NP // 4

    def body(xb, bb, mat, out):
        xv = xb[...]
        bcol = bb[...]
        oh = (lax.broadcasted_iota(jnp.int32, (BN, NB), 1)
              == jnp.broadcast_to(bcol, (BN, NB))).astype(F32)
        pos = _dot(oh, mat[...], ((1,), (0,)),
                   precision=lax.Precision.HIGHEST)
        cols = [xv[:, 0:1]]
        for k in range(3):
            col = (xv[:, 1:2] * pos[:, k:k + 1]
                   + xv[:, 2:3] * pos[:, 3 + k:4 + k]
                   + xv[:, 3:4] * pos[:, 6 + k:7 + k])
            cols.append(col)
        cols.append(jnp.zeros((BN, GW - 4), F32))
        out[...] = jnp.concatenate(cols, axis=1)

    return pl.pallas_call(
        body,
        grid=(4,),
        in_specs=[
            pl.BlockSpec((BN, 4), lambda i: (i, 0)),
            pl.BlockSpec((BN, 1), lambda i: (i, 0)),
            pl.BlockSpec((NB, 9), lambda i: (0, 0)),
        ],
        out_specs=pl.BlockSpec((BN, GW), lambda i: (i, 0)),
        out_shape=jax.ShapeDtypeStruct((NP, GW), F32),
    )(x_p, batch_2d, mat_flat)


@functools.lru_cache(maxsize=None)
def _make_conv(ic, oc, BE):
    F = ic * oc
    grid = EP // BE

    def body(hs, hd, hsi, Wa, ba, Wb, bb, out):
        ew = (hd[...] - hs[...])[:, 1:4]
        e1 = _leaky_silu(_dot(ew, Wa[...], ((1,), (1,))) + ba[...], 0.05)
        e2 = _leaky_silu(_dot(e1, Wb[...], ((1,), (1,))) + bb[...], 0.05)
        h = hsi[...]
        acc = h[:, 0:1] * e2[:, 0:oc]
        for i in range(1, ic):
            acc = acc + h[:, i:i + 1] * e2[:, i * oc:(i + 1) * oc]
        if GW > oc:
            acc = jnp.concatenate(
                [acc, jnp.zeros((BE, GW - oc), F32)], axis=1)
        out[...] = acc

    return pl.pallas_call(
        body,
        grid=(grid,),
        in_specs=[
            pl.BlockSpec((BE, GW), lambda i: (i, 0)),
            pl.BlockSpec((BE, GW), lambda i: (i, 0)),
            pl.BlockSpec((BE, GW), lambda i: (i, 0)),
            pl.BlockSpec((64, 3), lambda i: (0, 0)),
            pl.BlockSpec((1, 64), lambda i: (0, 0)),
            pl.BlockSpec((F, 64), lambda i: (0, 0)),
            pl.BlockSpec((1, F), lambda i: (0, 0)),
        ],
        out_specs=pl.BlockSpec((BE, GW), lambda i: (i, 0)),
        out_shape=jax.ShapeDtypeStruct((EP, GW), F32),
    )


@functools.lru_cache(maxsize=None)
def _make_node(ic, oc):
    BN = NP // 4

    def body(agg, hin, root, bias, W1, b1, W2, b2, out):
        asum = agg[...][:, :oc]
        h = hin[...][:, :ic]
        z = _leaky_silu(asum + _dot(h, root[...], ((1,), (1,)))
                        + bias[...], 0.1)
        val = _leaky_silu(_dot(_leaky_silu(_dot(z, W1[...], ((1,), (1,)))
                                           + b1[...], 0.05),
                               W2[...], ((1,), (1,))) + b2[...], 0.05) + z
        if GW > oc:
            val = jnp.concatenate(
                [val, jnp.zeros((BN, GW - oc), F32)], axis=1)
        out[...] = val

    return pl.pallas_call(
        body,
        grid=(4,),
        in_specs=[
            pl.BlockSpec((BN, GW), lambda i: (i, 0)),
            pl.BlockSpec((BN, GW), lambda i: (i, 0)),
            pl.BlockSpec((oc, ic), lambda i: (0, 0)),
            pl.BlockSpec((1, oc), lambda i: (0, 0)),
            pl.BlockSpec((128, oc), lambda i: (0, 0)),
            pl.BlockSpec((1, 128), lambda i: (0, 0)),
            pl.BlockSpec((oc, 128), lambda i: (0, 0)),
            pl.BlockSpec((1, oc), lambda i: (0, 0)),
        ],
        out_specs=pl.BlockSpec((BN, GW), lambda i: (i, 0)),
        out_shape=jax.ShapeDtypeStruct((NP, GW), F32),
    )


def _readout_call(h3, batch_2d, fcW, fcb):
    BN = NP // 4

    def body(h, bb, W, b, out, g):
        i = pl.program_id(0)

        @pl.when(i == 0)
        def _():
            g[...] = jnp.zeros((NB, 128), F32)

        bcol = bb[...]
        oh = (lax.broadcasted_iota(jnp.int32, (BN, NB), 1)
              == jnp.broadcast_to(bcol, (BN, NB))).astype(F32)
        g[...] += _dot(oh, h[...], ((0,), (0,)),
                       precision=lax.Precision.HIGHEST)
        val = (jnp.sum(g[...] * W[...], axis=1, keepdims=True)
               + jnp.broadcast_to(b[...], (NB, 1)))
        out[...] = -_leaky_silu(val, 0.1)

    return pl.pallas_call(
        body,
        grid=(4,),
        in_specs=[
            pl.BlockSpec((BN, GW), lambda i: (i, 0)),
            pl.BlockSpec((BN, 1), lambda i: (i, 0)),
            pl.BlockSpec((1, 128), lambda i: (0, 0)),
            pl.BlockSpec((1, 1), lambda i: (0, 0)),
        ],
        out_specs=pl.BlockSpec((NB, 1), lambda i: (0, 0)),
        out_shape=jax.ShapeDtypeStruct((NB, 1), F32),
        scratch_shapes=[pltpu.VMEM((NB, 128), F32)],
    )(h3, batch_2d, fcW, fcb)


def kernel(x, edge_index, matrix, batch, params):
    p = params
    i32 = jnp.int32
    src = edge_index[0]
    dst = edge_index[1]
    src_r = jnp.concatenate(
        [src, jnp.zeros((EP - E,), i32)]).reshape(2, 16, NCH, CH)
    dst_p = jnp.concatenate([dst, jnp.full((EP - E,), N, i32)])
    dst_r = dst_p.reshape(2, 16, NCH, CH)
    x_p = jnp.concatenate([x, jnp.zeros((NP - N, 4), F32)], axis=0)
    batch_2d = jnp.concatenate(
        [batch, jnp.full((NP - N,), NB, i32)]).reshape(NP, 1)
    mat_flat = matrix.reshape(NB, 9)

    h0 = _prep_call(x_p, batch_2d, mat_flat)

    gather = _make_gather()
    lists, cnts = _make_bin()(dst_p)
    scatter_b = _make_scatter()

    def scatter(msg, _):
        return scatter_b(msg, lists, cnts)

    hs = gather(h0, src_r)
    hd = gather(h0, dst_r)

    def r2(v):
        return v.reshape(1, -1)

    msg1 = _make_conv(4, 8, 1024)(hs, hd, hs, p['c1_Wa'], r2(p['c1_ba']),
                                  p['c1_Wb'], r2(p['c1_bb']))
    agg1 = scatter(msg1, dst_r)
    h1 = _make_node(4, 8)(
        agg1, h0, p['c1_root'], r2(p['c1_bias']),
        p['il1_W1'], r2(p['il1_b1']), p['il1_W2'], r2(p['il1_b2']))

    hs8 = gather(h1, src_r)
    msg2 = _make_conv(8, 64, 1024)(hs, hd, hs8, p['c2_Wa'], r2(p['c2_ba']),
                                   p['c2_Wb'], r2(p['c2_bb']))
    agg2 = scatter(msg2, dst_r)
    h2 = _make_node(8, 64)(
        agg2, h1, p['c2_root'], r2(p['c2_bias']),
        p['il2_W1'], r2(p['il2_b1']), p['il2_W2'], r2(p['il2_b2']))

    hs64 = gather(h2, src_r)
    msg3 = _make_conv(64, 128, 256)(hs, hd, hs64, p['c3_Wa'], r2(p['c3_ba']),
                                    p['c3_Wb'], r2(p['c3_bb']))
    agg3 = scatter(msg3, dst_r)
    h3 = _make_node(64, 128)(
        agg3, h2, p['c3_root'], r2(p['c3_bias']),
        p['il3_W1'], r2(p['il3_b1']), p['il3_W2'], r2(p['il3_b2']))

    return _readout_call(h3, batch_2d, r2(p['fc1_W']), r2(p['fc1_b']))


# speedup vs baseline: 1.7372x; 1.0418x over previous
import functools

import jax
import jax.numpy as jnp
from jax import lax
from jax.experimental import pallas as pl
from jax.experimental.pallas import tpu as pltpu
from jax.experimental.pallas import tpu_sc as plsc

N = 10000
NP = 10240
E = 50000
EP = 50176
TPW = 1568
NCH = 14
CH = 112
BPT = NP // 32
EPL = EP + CH
NB = 16
GW = 128
F32 = jnp.float32


def _leaky_silu(v, alpha):
    return v * jax.nn.sigmoid(v) + alpha * v


@functools.lru_cache(maxsize=None)
def _make_gather():
    mesh = plsc.VectorSubcoreMesh(
        core_axis_name="c", subcore_axis_name="s", num_cores=2)

    @functools.partial(
        pl.kernel,
        out_type=jax.ShapeDtypeStruct((EP, GW), F32),
        mesh=mesh,
        scratch_types=[
            pltpu.VMEM((NCH, CH), jnp.int32),
            pltpu.VMEM((CH, GW), F32),
            pltpu.VMEM((CH, GW), F32),
            pltpu.SemaphoreType.DMA,
            pltpu.SemaphoreType.DMA,
        ],
    )
    def gk(table_hbm, idx_hbm, out_hbm, idx_v, buf0, buf1, sem0, sem1):
        c = lax.axis_index("c")
        s = lax.axis_index("s")
        base = (c * 16 + s) * TPW
        pltpu.sync_copy(idx_hbm.at[c, s], idx_v)
        bufs = (buf0, buf1)
        sems = (sem0, sem1)
        cps = [None, None]
        for j in range(NCH):
            k = j % 2
            if cps[k] is not None:
                cps[k].wait()
                pltpu.sync_copy(bufs[k], out_hbm.at[pl.ds(base + (j - 2) * CH, CH)])
            cps[k] = pltpu.async_copy(
                table_hbm.at[idx_v.at[j]], bufs[k], sems[k])
        for j in range(NCH - 2, NCH):
            k = j % 2
            cps[k].wait()
            pltpu.sync_copy(bufs[k], out_hbm.at[pl.ds(base + j * CH, CH)])

    return gk


@functools.lru_cache(maxsize=None)
def _make_bin():
    mesh = plsc.VectorSubcoreMesh(
        core_axis_name="c", subcore_axis_name="s", num_cores=2)

    @functools.partial(
        pl.kernel,
        out_type=(jax.ShapeDtypeStruct((32 * EPL,), jnp.int32),
                  jax.ShapeDtypeStruct((32 * 16,), jnp.int32)),
        mesh=mesh,
        compiler_params=pltpu.CompilerParams(needs_layout_passes=False),
        scratch_types=[
            pltpu.VMEM((TPW,), jnp.int32),
            pltpu.VMEM((EPL,), jnp.int32),
            pltpu.VMEM((16,), jnp.int32),
        ],
    )
    def bk(dst_hbm, lists_hbm, cnts_hbm, idx_v, list_v, cnt_v):
        c = lax.axis_index("c")
        s = lax.axis_index("s")
        w = c * 16 + s
        rbase = w * BPT
        cnt = jnp.int32(0)
        for ch in range(EP // TPW):
            pltpu.sync_copy(dst_hbm.at[pl.ds(ch * TPW, TPW)], idx_v)

            def scan_g(g, cnt):
                v = idx_v[pl.ds(g * 16, 16)]
                lv = v - rbase
                m = (lv >= 0) & (lv < BPT)
                eid = (lax.broadcasted_iota(jnp.int32, (16,), 0)
                       + (ch * TPW + g * 16))
                packed = eid * 512 + jnp.where(m, lv, 0)
                mc = jnp.cumsum(m.astype(jnp.int32))
                plsc.store_scatter(list_v, [cnt + mc - 1], packed, mask=m)
                return cnt + mc[15]

            cnt = lax.fori_loop(0, TPW // 16, scan_g, cnt)
        dummy = jnp.full((16,), BPT, jnp.int32)
        for e in range(CH // 16):
            list_v[pl.ds(cnt + e * 16, 16)] = dummy
        cnt_v[...] = jnp.full((16,), cnt, jnp.int32)
        pltpu.sync_copy(list_v, lists_hbm.at[pl.ds(w * EPL, EPL)])
        pltpu.sync_copy(cnt_v, cnts_hbm.at[pl.ds(w * 16, 16)])

    return bk


@functools.lru_cache(maxsize=None)
def _make_scatter():
    mesh = plsc.VectorSubcoreMesh(
        core_axis_name="c", subcore_axis_name="s", num_cores=2)

    @functools.partial(
        pl.kernel,
        out_type=jax.ShapeDtypeStruct((NP, GW), F32),
        mesh=mesh,
        compiler_params=pltpu.CompilerParams(needs_layout_passes=False),
        scratch_types=[
            pltpu.VMEM((16,), jnp.int32),
            pltpu.VMEM((CH,), jnp.int32),
            pltpu.VMEM((CH,), jnp.int32),
            pltpu.VMEM((CH,), jnp.int32),
            pltpu.VMEM((CH,), jnp.int32),
            pltpu.VMEM((CH,), jnp.int32),
            pltpu.VMEM((CH, GW), F32),
            pltpu.VMEM((CH, GW), F32),
            pltpu.VMEM((BPT + 8, GW), F32),
            pltpu.SemaphoreType.DMA,
            pltpu.SemaphoreType.DMA,
        ],
    )
    def sk(msg_hbm, lists_hbm, cnts_hbm, out_hbm, cnt_v, pk_v, eid_v, lv_v,
           eid_v2, lv_v2, msg_v, msg_v2, acc, sem, sem2):
        c = lax.axis_index("c")
        s = lax.axis_index("s")
        w = c * 16 + s

        def zero_row(i, carry):
            for k in range(GW // 16):
                acc[i, pl.ds(k * 16, 16)] = jnp.zeros((16,), F32)
            return carry

        lax.fori_loop(0, BPT + 8, zero_row, 0)

        pltpu.sync_copy(cnts_hbm.at[pl.ds(w * 16, 16)], cnt_v)
        cnt = cnt_v[...][0]
        nch = lax.div(cnt + (CH - 1), jnp.int32(CH))

        eid_b = (eid_v, eid_v2)
        lv_b = (lv_v, lv_v2)
        msg_b = (msg_v, msg_v2)
        sem_b = (sem, sem2)

        def load_unpack(jj, b):
            pltpu.sync_copy(lists_hbm.at[pl.ds(w * EPL + jj * CH, CH)], pk_v)

            def unpack(g, carry2):
                pk = pk_v[pl.ds(g * 16, 16)]
                eid_b[b][pl.ds(g * 16, 16)] = lax.shift_right_logical(pk, 9)
                lv_b[b][pl.ds(g * 16, 16)] = lax.bitwise_and(
                    pk, jnp.full((16,), 511, jnp.int32))
                return carry2

            lax.fori_loop(0, CH // 16, unpack, 0)

        def accumulate(b):
            def edge_group(g, carry2):
                lv16 = lv_b[b][pl.ds(g * 16, 16)]
                for e2 in range(16):
                    lv = lv16[e2]
                    row = g * 16 + e2
                    for k in range(GW // 16):
                        plsc.addupdate(acc.at[lv, pl.ds(k * 16, 16)],
                                       msg_b[b][row, pl.ds(k * 16, 16)])
                return carry2

            lax.fori_loop(0, CH // 16, edge_group, 0)

        @pl.when(nch > 0)
        def _():
            load_unpack(jnp.int32(0), 0)
            pltpu.async_copy(msg_hbm.at[eid_b[0]], msg_b[0], sem_b[0])

        def pair(i, carry):
            for b in range(2):
                jj = i * 2 + b
                nb = 1 - b

                @pl.when(jj < nch)
                def _():
                    @pl.when(jj + 1 < nch)
                    def _():
                        load_unpack(jj + 1, nb)
                        pltpu.async_copy(
                            msg_hbm.at[eid_b[nb]], msg_b[nb], sem_b[nb])

                    pltpu.make_async_copy(
                        msg_hbm.at[eid_b[b]], msg_b[b], sem_b[b]).wait()
                    accumulate(b)

            return carry

        lax.fori_loop(0, lax.div(nch + 1, jnp.int32(2)), pair, 0)
        pltpu.sync_copy(acc.at[pl.ds(0, BPT)],
                        out_hbm.at[pl.ds(w * BPT, BPT)])

    return sk


def _dot(a, b, dims, precision=None):
    return lax.dot_general(a, b, (dims, ((), ())),
                           preferred_element_type=F32, precision=precision)


def _prep_call(x_p, batch_2d, mat_flat):
    BN = NP // 4

    def body(xb, bb, mat, out):
        xv = xb[...]
        bcol = bb[...]
        oh = (lax.broadcasted_iota(jnp.int32, (BN, NB), 1)
              == jnp.broadcast_to(bcol, (BN, NB))).astype(F32)
        pos = _dot(oh, mat[...], ((1,), (0,)),
                   precision=lax.Precision.HIGHEST)
        cols = [xv[:, 0:1]]
        for k in range(3):
            col = (xv[:, 1:2] * pos[:, k:k + 1]
                   + xv[:, 2:3] * pos[:, 3 + k:4 + k]
                   + xv[:, 3:4] * pos[:, 6 + k:7 + k])
            cols.append(col)
        cols.append(jnp.zeros((BN, GW - 4), F32))
        out[...] = jnp.concatenate(cols, axis=1)

    return pl.pallas_call(
        body,
        grid=(4,),
        in_specs=[
            pl.BlockSpec((BN, 4), lambda i: (i, 0)),
            pl.BlockSpec((BN, 1), lambda i: (i, 0)),
            pl.BlockSpec((NB, 9), lambda i: (0, 0)),
        ],
        out_specs=pl.BlockSpec((BN, GW), lambda i: (i, 0)),
        out_shape=jax.ShapeDtypeStruct((NP, GW), F32),
    )(x_p, batch_2d, mat_flat)


@functools.lru_cache(maxsize=None)
def _make_conv(ic, oc, BE, IG):
    F = ic * oc
    grid = EP // BE
    ngrp = ic // IG

    def body(hs, hd, hsi, Wa, ba, Wb, bb, out):
        ew = (hd[...] - hs[...])[:, 1:4]
        e1 = _leaky_silu(_dot(ew, Wa[...], ((1,), (1,))) + ba[...], 0.05)
        h = hsi[...]
        acc = None
        for grp in range(ngrp):
            lo = grp * IG * oc
            z = (_dot(e1, Wb[lo:lo + IG * oc, :], ((1,), (1,)))
                 + bb[:, lo:lo + IG * oc])
            a = _leaky_silu(z, 0.05)
            for i2 in range(IG):
                t = h[:, grp * IG + i2:grp * IG + i2 + 1] * (
                    a[:, i2 * oc:(i2 + 1) * oc])
                acc = t if acc is None else acc + t
        if GW > oc:
            acc = jnp.concatenate(
                [acc, jnp.zeros((BE, GW - oc), F32)], axis=1)
        out[...] = acc

    return pl.pallas_call(
        body,
        grid=(grid,),
        in_specs=[
            pl.BlockSpec((BE, GW), lambda i: (i, 0)),
            pl.BlockSpec((BE, GW), lambda i: (i, 0)),
            pl.BlockSpec((BE, GW), lambda i: (i, 0)),
            pl.BlockSpec((64, 3), lambda i: (0, 0)),
            pl.BlockSpec((1, 64), lambda i: (0, 0)),
            pl.BlockSpec((F, 64), lambda i: (0, 0)),
            pl.BlockSpec((1, F), lambda i: (0, 0)),
        ],
        out_specs=pl.BlockSpec((BE, GW), lambda i: (i, 0)),
        out_shape=jax.ShapeDtypeStruct((EP, GW), F32),
    )


@functools.lru_cache(maxsize=None)
def _make_node(ic, oc):
    BN = NP // 4

    def body(agg, hin, root, bias, W1, b1, W2, b2, out):
        asum = agg[...][:, :oc]
        h = hin[...][:, :ic]
        z = _leaky_silu(asum + _dot(h, root[...], ((1,), (1,)))
                        + bias[...], 0.1)
        val = _leaky_silu(_dot(_leaky_silu(_dot(z, W1[...], ((1,), (1,)))
                                           + b1[...], 0.05),
                               W2[...], ((1,), (1,))) + b2[...], 0.05) + z
        if GW > oc:
            val = jnp.concatenate(
                [val, jnp.zeros((BN, GW - oc), F32)], axis=1)
        out[...] = val

    return pl.pallas_call(
        body,
        grid=(4,),
        in_specs=[
            pl.BlockSpec((BN, GW), lambda i: (i, 0)),
            pl.BlockSpec((BN, GW), lambda i: (i, 0)),
            pl.BlockSpec((oc, ic), lambda i: (0, 0)),
            pl.BlockSpec((1, oc), lambda i: (0, 0)),
            pl.BlockSpec((128, oc), lambda i: (0, 0)),
            pl.BlockSpec((1, 128), lambda i: (0, 0)),
            pl.BlockSpec((oc, 128), lambda i: (0, 0)),
            pl.BlockSpec((1, oc), lambda i: (0, 0)),
        ],
        out_specs=pl.BlockSpec((BN, GW), lambda i: (i, 0)),
        out_shape=jax.ShapeDtypeStruct((NP, GW), F32),
    )


def _readout_call(h3, batch_2d, fcW, fcb):
    BN = NP // 4

    def body(h, bb, W, b, out, g):
        i = pl.program_id(0)

        @pl.when(i == 0)
        def _():
            g[...] = jnp.zeros((NB, 128), F32)

        bcol = bb[...]
        oh = (lax.broadcasted_iota(jnp.int32, (BN, NB), 1)
              == jnp.broadcast_to(bcol, (BN, NB))).astype(F32)
        g[...] += _dot(oh, h[...], ((0,), (0,)),
                       precision=lax.Precision.HIGHEST)
        val = (jnp.sum(g[...] * W[...], axis=1, keepdims=True)
               + jnp.broadcast_to(b[...], (NB, 1)))
        out[...] = -_leaky_silu(val, 0.1)

    return pl.pallas_call(
        body,
        grid=(4,),
        in_specs=[
            pl.BlockSpec((BN, GW), lambda i: (i, 0)),
            pl.BlockSpec((BN, 1), lambda i: (i, 0)),
            pl.BlockSpec((1, 128), lambda i: (0, 0)),
            pl.BlockSpec((1, 1), lambda i: (0, 0)),
        ],
        out_specs=pl.BlockSpec((NB, 1), lambda i: (0, 0)),
        out_shape=jax.ShapeDtypeStruct((NB, 1), F32),
        scratch_shapes=[pltpu.VMEM((NB, 128), F32)],
    )(h3, batch_2d, fcW, fcb)


def kernel(x, edge_index, matrix, batch, params):
    p = params
    i32 = jnp.int32
    src = edge_index[0]
    dst = edge_index[1]
    src_r = jnp.concatenate(
        [src, jnp.zeros((EP - E,), i32)]).reshape(2, 16, NCH, CH)
    dst_p = jnp.concatenate([dst, jnp.full((EP - E,), N, i32)])
    dst_r = dst_p.reshape(2, 16, NCH, CH)
    x_p = jnp.concatenate([x, jnp.zeros((NP - N, 4), F32)], axis=0)
    batch_2d = jnp.concatenate(
        [batch, jnp.full((NP - N,), NB, i32)]).reshape(NP, 1)
    mat_flat = matrix.reshape(NB, 9)

    h0 = _prep_call(x_p, batch_2d, mat_flat)

    gather = _make_gather()
    lists, cnts = _make_bin()(dst_p)
    scatter_b = _make_scatter()

    def scatter(msg, _):
        return scatter_b(msg, lists, cnts)

    hs = gather(h0, src_r)
    hd = gather(h0, dst_r)

    def r2(v):
        return v.reshape(1, -1)

    msg1 = _make_conv(4, 8, 1024, 4)(hs, hd, hs, p['c1_Wa'], r2(p['c1_ba']),
                                  p['c1_Wb'], r2(p['c1_bb']))
    agg1 = scatter(msg1, dst_r)
    h1 = _make_node(4, 8)(
        agg1, h0, p['c1_root'], r2(p['c1_bias']),
        p['il1_W1'], r2(p['il1_b1']), p['il1_W2'], r2(p['il1_b2']))

    hs8 = gather(h1, src_r)
    msg2 = _make_conv(8, 64, 1024, 8)(hs, hd, hs8, p['c2_Wa'], r2(p['c2_ba']),
                                   p['c2_Wb'], r2(p['c2_bb']))
    agg2 = scatter(msg2, dst_r)
    h2 = _make_node(8, 64)(
        agg2, h1, p['c2_root'], r2(p['c2_bias']),
        p['il2_W1'], r2(p['il2_b1']), p['il2_W2'], r2(p['il2_b2']))

    hs64 = gather(h2, src_r)
    msg3 = _make_conv(64, 128, 512, 16)(hs, hd, hs64, p['c3_Wa'], r2(p['c3_ba']),
                                    p['c3_Wb'], r2(p['c3_bb']))
    agg3 = scatter(msg3, dst_r)
    h3 = _make_node(64, 128)(
        agg3, h2, p['c3_root'], r2(p['c3_bias']),
        p['il3_W1'], r2(p['il3_b1']), p['il3_W2'], r2(p['il3_b2']))

    return _readout_call(h3, batch_2d, r2(p['fc1_W']), r2(p['fc1_b']))


# speedup vs baseline: 1.9039x; 1.0960x over previous
import functools

import jax
import jax.numpy as jnp
from jax import lax
from jax.experimental import pallas as pl
from jax.experimental.pallas import tpu as pltpu
from jax.experimental.pallas import tpu_sc as plsc

N = 10000
NP = 10240
E = 50000
EP = 50176
TPW = 1568
NCH = 14
CH = 112
BPT = NP // 32
EPL = EP + CH
NB = 16
GW = 128
F32 = jnp.float32


def _leaky_silu(v, alpha):
    return v * (0.5 * jnp.tanh(0.5 * v) + (0.5 + alpha))


@functools.lru_cache(maxsize=None)
def _make_gather():
    mesh = plsc.VectorSubcoreMesh(
        core_axis_name="c", subcore_axis_name="s", num_cores=2)

    @functools.partial(
        pl.kernel,
        out_type=jax.ShapeDtypeStruct((EP, GW), F32),
        mesh=mesh,
        scratch_types=[
            pltpu.VMEM((NCH, CH), jnp.int32),
            pltpu.VMEM((CH, GW), F32),
            pltpu.VMEM((CH, GW), F32),
            pltpu.SemaphoreType.DMA,
            pltpu.SemaphoreType.DMA,
        ],
    )
    def gk(table_hbm, idx_hbm, out_hbm, idx_v, buf0, buf1, sem0, sem1):
        c = lax.axis_index("c")
        s = lax.axis_index("s")
        base = (c * 16 + s) * TPW
        pltpu.sync_copy(idx_hbm.at[c, s], idx_v)
        bufs = (buf0, buf1)
        sems = (sem0, sem1)
        cps = [None, None]
        for j in range(NCH):
            k = j % 2
            if cps[k] is not None:
                cps[k].wait()
                pltpu.sync_copy(bufs[k], out_hbm.at[pl.ds(base + (j - 2) * CH, CH)])
            cps[k] = pltpu.async_copy(
                table_hbm.at[idx_v.at[j]], bufs[k], sems[k])
        for j in range(NCH - 2, NCH):
            k = j % 2
            cps[k].wait()
            pltpu.sync_copy(bufs[k], out_hbm.at[pl.ds(base + j * CH, CH)])

    return gk


@functools.lru_cache(maxsize=None)
def _make_bin():
    mesh = plsc.VectorSubcoreMesh(
        core_axis_name="c", subcore_axis_name="s", num_cores=2)

    @functools.partial(
        pl.kernel,
        out_type=(jax.ShapeDtypeStruct((32 * EPL,), jnp.int32),
                  jax.ShapeDtypeStruct((32 * 16,), jnp.int32)),
        mesh=mesh,
        compiler_params=pltpu.CompilerParams(needs_layout_passes=False),
        scratch_types=[
            pltpu.VMEM((TPW,), jnp.int32),
            pltpu.VMEM((EPL,), jnp.int32),
            pltpu.VMEM((16,), jnp.int32),
        ],
    )
    def bk(dst_hbm, lists_hbm, cnts_hbm, idx_v, list_v, cnt_v):
        c = lax.axis_index("c")
        s = lax.axis_index("s")
        w = c * 16 + s
        rbase = w * BPT
        cnt = jnp.int32(0)
        for ch in range(EP // TPW):
            pltpu.sync_copy(dst_hbm.at[pl.ds(ch * TPW, TPW)], idx_v)

            def scan_g(g, cnt):
                v = idx_v[pl.ds(g * 16, 16)]
                lv = v - rbase
                m = (lv >= 0) & (lv < BPT)
                eid = (lax.broadcasted_iota(jnp.int32, (16,), 0)
                       + (ch * TPW + g * 16))
                packed = eid * 512 + jnp.where(m, lv, 0)
                mc = jnp.cumsum(m.astype(jnp.int32))
                plsc.store_scatter(list_v, [cnt + mc - 1], packed, mask=m)
                return cnt + mc[15]

            cnt = lax.fori_loop(0, TPW // 16, scan_g, cnt)
        dummy = jnp.full((16,), BPT, jnp.int32)
        for e in range(CH // 16):
            list_v[pl.ds(cnt + e * 16, 16)] = dummy
        cnt_v[...] = jnp.full((16,), cnt, jnp.int32)
        pltpu.sync_copy(list_v, lists_hbm.at[pl.ds(w * EPL, EPL)])
        pltpu.sync_copy(cnt_v, cnts_hbm.at[pl.ds(w * 16, 16)])

    return bk


@functools.lru_cache(maxsize=None)
def _make_scatter():
    mesh = plsc.VectorSubcoreMesh(
        core_axis_name="c", subcore_axis_name="s", num_cores=2)

    @functools.partial(
        pl.kernel,
        out_type=jax.ShapeDtypeStruct((NP, GW), F32),
        mesh=mesh,
        compiler_params=pltpu.CompilerParams(needs_layout_passes=False),
        scratch_types=[
            pltpu.VMEM((16,), jnp.int32),
            pltpu.VMEM((CH,), jnp.int32),
            pltpu.VMEM((CH,), jnp.int32),
            pltpu.VMEM((CH,), jnp.int32),
            pltpu.VMEM((CH,), jnp.int32),
            pltpu.VMEM((CH,), jnp.int32),
            pltpu.VMEM((CH, GW), F32),
            pltpu.VMEM((CH, GW), F32),
            pltpu.VMEM((BPT + 8, GW), F32),
            pltpu.SemaphoreType.DMA,
            pltpu.SemaphoreType.DMA,
        ],
    )
    def sk(msg_hbm, lists_hbm, cnts_hbm, out_hbm, cnt_v, pk_v, eid_v, lv_v,
           eid_v2, lv_v2, msg_v, msg_v2, acc, sem, sem2):
        c = lax.axis_index("c")
        s = lax.axis_index("s")
        w = c * 16 + s

        def zero_row(i, carry):
            for k in range(GW // 16):
                acc[i, pl.ds(k * 16, 16)] = jnp.zeros((16,), F32)
            return carry

        lax.fori_loop(0, BPT + 8, zero_row, 0)

        pltpu.sync_copy(cnts_hbm.at[pl.ds(w * 16, 16)], cnt_v)
        cnt = cnt_v[...][0]
        nch = lax.div(cnt + (CH - 1), jnp.int32(CH))

        eid_b = (eid_v, eid_v2)
        lv_b = (lv_v, lv_v2)
        msg_b = (msg_v, msg_v2)
        sem_b = (sem, sem2)

        def load_unpack(jj, b):
            pltpu.sync_copy(lists_hbm.at[pl.ds(w * EPL + jj * CH, CH)], pk_v)

            def unpack(g, carry2):
                pk = pk_v[pl.ds(g * 16, 16)]
                eid_b[b][pl.ds(g * 16, 16)] = lax.shift_right_logical(pk, 9)
                lv_b[b][pl.ds(g * 16, 16)] = lax.bitwise_and(
                    pk, jnp.full((16,), 511, jnp.int32))
                return carry2

            lax.fori_loop(0, CH // 16, unpack, 0)

        def accumulate(b):
            def edge_group(g, carry2):
                lv16 = lv_b[b][pl.ds(g * 16, 16)]
                for e2 in range(16):
                    lv = lv16[e2]
                    row = g * 16 + e2
                    for k in range(GW // 16):
                        plsc.addupdate(acc.at[lv, pl.ds(k * 16, 16)],
                                       msg_b[b][row, pl.ds(k * 16, 16)])
                return carry2

            lax.fori_loop(0, CH // 16, edge_group, 0)

        @pl.when(nch > 0)
        def _():
            load_unpack(jnp.int32(0), 0)
            pltpu.async_copy(msg_hbm.at[eid_b[0]], msg_b[0], sem_b[0])

        def pair(i, carry):
            for b in range(2):
                jj = i * 2 + b
                nb = 1 - b

                @pl.when(jj < nch)
                def _():
                    @pl.when(jj + 1 < nch)
                    def _():
                        load_unpack(jj + 1, nb)
                        pltpu.async_copy(
                            msg_hbm.at[eid_b[nb]], msg_b[nb], sem_b[nb])

                    pltpu.make_async_copy(
                        msg_hbm.at[eid_b[b]], msg_b[b], sem_b[b]).wait()
                    accumulate(b)

            return carry

        lax.fori_loop(0, lax.div(nch + 1, jnp.int32(2)), pair, 0)
        pltpu.sync_copy(acc.at[pl.ds(0, BPT)],
                        out_hbm.at[pl.ds(w * BPT, BPT)])

    return sk


def _dot(a, b, dims, precision=None):
    return lax.dot_general(a, b, (dims, ((), ())),
                           preferred_element_type=F32, precision=precision)


def _prep_call(x_p, batch_2d, mat_flat):
    BN = NP // 4

    def body(xb, bb, mat, out):
        xv = xb[...]
        bcol = bb[...]
        oh = (lax.broadcasted_iota(jnp.int32, (BN, NB), 1)
              == jnp.broadcast_to(bcol, (BN, NB))).astype(F32)
        pos = _dot(oh, mat[...], ((1,), (0,)),
                   precision=lax.Precision.HIGHEST)
        cols = [xv[:, 0:1]]
        for k in range(3):
            col = (xv[:, 1:2] * pos[:, k:k + 1]
                   + xv[:, 2:3] * pos[:, 3 + k:4 + k]
                   + xv[:, 3:4] * pos[:, 6 + k:7 + k])
            cols.append(col)
        cols.append(jnp.zeros((BN, GW - 4), F32))
        out[...] = jnp.concatenate(cols, axis=1)

    return pl.pallas_call(
        body,
        grid=(4,),
        in_specs=[
            pl.BlockSpec((BN, 4), lambda i: (i, 0)),
            pl.BlockSpec((BN, 1), lambda i: (i, 0)),
            pl.BlockSpec((NB, 9), lambda i: (0, 0)),
        ],
        out_specs=pl.BlockSpec((BN, GW), lambda i: (i, 0)),
        out_shape=jax.ShapeDtypeStruct((NP, GW), F32),
    )(x_p, batch_2d, mat_flat)


@functools.lru_cache(maxsize=None)
def _make_conv(ic, oc, BE, IG):
    F = ic * oc
    grid = EP // BE
    ngrp = ic // IG

    def body(hs, hd, hsi, Wa, ba, Wb, bb, out):
        ew = (hd[...] - hs[...])[:, 1:4]
        e1 = _leaky_silu(_dot(ew, Wa[...], ((1,), (1,))) + ba[...], 0.05)
        h = hsi[...]
        acc = None
        for grp in range(ngrp):
            lo = grp * IG * oc
            z = (_dot(e1, Wb[lo:lo + IG * oc, :], ((1,), (1,)))
                 + bb[:, lo:lo + IG * oc])
            a = _leaky_silu(z, 0.05)
            for i2 in range(IG):
                t = h[:, grp * IG + i2:grp * IG + i2 + 1] * (
                    a[:, i2 * oc:(i2 + 1) * oc])
                acc = t if acc is None else acc + t
        if GW > oc:
            acc = jnp.concatenate(
                [acc, jnp.zeros((BE, GW - oc), F32)], axis=1)
        out[...] = acc

    return pl.pallas_call(
        body,
        grid=(grid,),
        in_specs=[
            pl.BlockSpec((BE, GW), lambda i: (i, 0)),
            pl.BlockSpec((BE, GW), lambda i: (i, 0)),
            pl.BlockSpec((BE, GW), lambda i: (i, 0)),
            pl.BlockSpec((64, 3), lambda i: (0, 0)),
            pl.BlockSpec((1, 64), lambda i: (0, 0)),
            pl.BlockSpec((F, 64), lambda i: (0, 0)),
            pl.BlockSpec((1, F), lambda i: (0, 0)),
        ],
        out_specs=pl.BlockSpec((BE, GW), lambda i: (i, 0)),
        out_shape=jax.ShapeDtypeStruct((EP, GW), F32),
    )


@functools.lru_cache(maxsize=None)
def _make_node(ic, oc):
    BN = NP // 4

    def body(agg, hin, root, bias, W1, b1, W2, b2, out):
        asum = agg[...][:, :oc]
        h = hin[...][:, :ic]
        z = _leaky_silu(asum + _dot(h, root[...], ((1,), (1,)))
                        + bias[...], 0.1)
        val = _leaky_silu(_dot(_leaky_silu(_dot(z, W1[...], ((1,), (1,)))
                                           + b1[...], 0.05),
                               W2[...], ((1,), (1,))) + b2[...], 0.05) + z
        if GW > oc:
            val = jnp.concatenate(
                [val, jnp.zeros((BN, GW - oc), F32)], axis=1)
        out[...] = val

    return pl.pallas_call(
        body,
        grid=(4,),
        in_specs=[
            pl.BlockSpec((BN, GW), lambda i: (i, 0)),
            pl.BlockSpec((BN, GW), lambda i: (i, 0)),
            pl.BlockSpec((oc, ic), lambda i: (0, 0)),
            pl.BlockSpec((1, oc), lambda i: (0, 0)),
            pl.BlockSpec((128, oc), lambda i: (0, 0)),
            pl.BlockSpec((1, 128), lambda i: (0, 0)),
            pl.BlockSpec((oc, 128), lambda i: (0, 0)),
            pl.BlockSpec((1, oc), lambda i: (0, 0)),
        ],
        out_specs=pl.BlockSpec((BN, GW), lambda i: (i, 0)),
        out_shape=jax.ShapeDtypeStruct((NP, GW), F32),
    )


def _readout_call(h3, batch_2d, fcW, fcb):
    BN = NP // 4

    def body(h, bb, W, b, out, g):
        i = pl.program_id(0)

        @pl.when(i == 0)
        def _():
            g[...] = jnp.zeros((NB, 128), F32)

        bcol = bb[...]
        oh = (lax.broadcasted_iota(jnp.int32, (BN, NB), 1)
              == jnp.broadcast_to(bcol, (BN, NB))).astype(F32)
        g[...] += _dot(oh, h[...], ((0,), (0,)),
                       precision=lax.Precision.HIGHEST)
        val = (jnp.sum(g[...] * W[...], axis=1, keepdims=True)
               + jnp.broadcast_to(b[...], (NB, 1)))
        out[...] = -_leaky_silu(val, 0.1)

    return pl.pallas_call(
        body,
        grid=(4,),
        in_specs=[
            pl.BlockSpec((BN, GW), lambda i: (i, 0)),
            pl.BlockSpec((BN, 1), lambda i: (i, 0)),
            pl.BlockSpec((1, 128), lambda i: (0, 0)),
            pl.BlockSpec((1, 1), lambda i: (0, 0)),
        ],
        out_specs=pl.BlockSpec((NB, 1), lambda i: (0, 0)),
        out_shape=jax.ShapeDtypeStruct((NB, 1), F32),
        scratch_shapes=[pltpu.VMEM((NB, 128), F32)],
    )(h3, batch_2d, fcW, fcb)


def kernel(x, edge_index, matrix, batch, params):
    p = params
    i32 = jnp.int32
    src = edge_index[0]
    dst = edge_index[1]
    src_r = jnp.concatenate(
        [src, jnp.zeros((EP - E,), i32)]).reshape(2, 16, NCH, CH)
    dst_p = jnp.concatenate([dst, jnp.full((EP - E,), N, i32)])
    dst_r = dst_p.reshape(2, 16, NCH, CH)
    x_p = jnp.concatenate([x, jnp.zeros((NP - N, 4), F32)], axis=0)
    batch_2d = jnp.concatenate(
        [batch, jnp.full((NP - N,), NB, i32)]).reshape(NP, 1)
    mat_flat = matrix.reshape(NB, 9)

    h0 = _prep_call(x_p, batch_2d, mat_flat)

    gather = _make_gather()
    lists, cnts = _make_bin()(dst_p)
    scatter_b = _make_scatter()

    def scatter(msg, _):
        return scatter_b(msg, lists, cnts)

    hs = gather(h0, src_r)
    hd = gather(h0, dst_r)

    def r2(v):
        return v.reshape(1, -1)

    msg1 = _make_conv(4, 8, 1024, 4)(hs, hd, hs, p['c1_Wa'], r2(p['c1_ba']),
                                  p['c1_Wb'], r2(p['c1_bb']))
    agg1 = scatter(msg1, dst_r)
    h1 = _make_node(4, 8)(
        agg1, h0, p['c1_root'], r2(p['c1_bias']),
        p['il1_W1'], r2(p['il1_b1']), p['il1_W2'], r2(p['il1_b2']))

    hs8 = gather(h1, src_r)
    msg2 = _make_conv(8, 64, 1024, 8)(hs, hd, hs8, p['c2_Wa'], r2(p['c2_ba']),
                                   p['c2_Wb'], r2(p['c2_bb']))
    agg2 = scatter(msg2, dst_r)
    h2 = _make_node(8, 64)(
        agg2, h1, p['c2_root'], r2(p['c2_bias']),
        p['il2_W1'], r2(p['il2_b1']), p['il2_W2'], r2(p['il2_b2']))

    hs64 = gather(h2, src_r)
    msg3 = _make_conv(64, 128, 512, 16)(hs, hd, hs64, p['c3_Wa'], r2(p['c3_ba']),
                                    p['c3_Wb'], r2(p['c3_bb']))
    agg3 = scatter(msg3, dst_r)
    h3 = _make_node(64, 128)(
        agg3, h2, p['c3_root'], r2(p['c3_bias']),
        p['il3_W1'], r2(p['il3_b1']), p['il3_W2'], r2(p['il3_b2']))

    return _readout_call(h3, batch_2d, r2(p['fc1_W']), r2(p['fc1_b']))


# speedup vs baseline: 2.1001x; 1.1031x over previous
import functools

import jax
import jax.numpy as jnp
from jax import lax
from jax.experimental import pallas as pl
from jax.experimental.pallas import tpu as pltpu
from jax.experimental.pallas import tpu_sc as plsc

N = 10000
NP = 10240
E = 50000
EP = 50176
TPW = 1568
NCH = 14
CH = 112
BPT = NP // 32
EPL = EP + CH
NB = 16
GW = 128
F32 = jnp.float32


def _leaky_silu(v, alpha):
    return v * (0.5 * jnp.tanh(0.5 * v) + (0.5 + alpha))


@functools.lru_cache(maxsize=None)
def _make_gather():
    mesh = plsc.VectorSubcoreMesh(
        core_axis_name="c", subcore_axis_name="s", num_cores=2)

    @functools.partial(
        pl.kernel,
        out_type=jax.ShapeDtypeStruct((EP, GW), F32),
        mesh=mesh,
        scratch_types=[
            pltpu.VMEM((NCH, CH), jnp.int32),
            pltpu.VMEM((CH, GW), F32),
            pltpu.VMEM((CH, GW), F32),
            pltpu.SemaphoreType.DMA,
            pltpu.SemaphoreType.DMA,
        ],
    )
    def gk(table_hbm, idx_hbm, out_hbm, idx_v, buf0, buf1, sem0, sem1):
        c = lax.axis_index("c")
        s = lax.axis_index("s")
        base = (c * 16 + s) * TPW
        pltpu.sync_copy(idx_hbm.at[c, s], idx_v)
        bufs = (buf0, buf1)
        sems = (sem0, sem1)
        cps = [None, None]
        for j in range(NCH):
            k = j % 2
            if cps[k] is not None:
                cps[k].wait()
                pltpu.sync_copy(bufs[k], out_hbm.at[pl.ds(base + (j - 2) * CH, CH)])
            cps[k] = pltpu.async_copy(
                table_hbm.at[idx_v.at[j]], bufs[k], sems[k])
        for j in range(NCH - 2, NCH):
            k = j % 2
            cps[k].wait()
            pltpu.sync_copy(bufs[k], out_hbm.at[pl.ds(base + j * CH, CH)])

    return gk


@functools.lru_cache(maxsize=None)
def _make_bin():
    mesh = plsc.VectorSubcoreMesh(
        core_axis_name="c", subcore_axis_name="s", num_cores=2)

    @functools.partial(
        pl.kernel,
        out_type=(jax.ShapeDtypeStruct((32 * EPL,), jnp.int32),
                  jax.ShapeDtypeStruct((32 * 16,), jnp.int32)),
        mesh=mesh,
        compiler_params=pltpu.CompilerParams(needs_layout_passes=False),
        scratch_types=[
            pltpu.VMEM((TPW,), jnp.int32),
            pltpu.VMEM((EPL,), jnp.int32),
            pltpu.VMEM((16,), jnp.int32),
        ],
    )
    def bk(dst_hbm, lists_hbm, cnts_hbm, idx_v, list_v, cnt_v):
        c = lax.axis_index("c")
        s = lax.axis_index("s")
        w = c * 16 + s
        rbase = w * BPT
        cnt = jnp.int32(0)
        for ch in range(EP // TPW):
            pltpu.sync_copy(dst_hbm.at[pl.ds(ch * TPW, TPW)], idx_v)

            def scan_g(g, cnt):
                v = idx_v[pl.ds(g * 16, 16)]
                lv = v - rbase
                m = (lv >= 0) & (lv < BPT)
                eid = (lax.broadcasted_iota(jnp.int32, (16,), 0)
                       + (ch * TPW + g * 16))
                packed = eid * 512 + jnp.where(m, lv, 0)
                mc = jnp.cumsum(m.astype(jnp.int32))
                plsc.store_scatter(list_v, [cnt + mc - 1], packed, mask=m)
                return cnt + mc[15]

            cnt = lax.fori_loop(0, TPW // 16, scan_g, cnt)
        dummy = jnp.full((16,), BPT, jnp.int32)
        for e in range(CH // 16):
            list_v[pl.ds(cnt + e * 16, 16)] = dummy
        cnt_v[...] = jnp.full((16,), cnt, jnp.int32)
        pltpu.sync_copy(list_v, lists_hbm.at[pl.ds(w * EPL, EPL)])
        pltpu.sync_copy(cnt_v, cnts_hbm.at[pl.ds(w * 16, 16)])

    return bk


@functools.lru_cache(maxsize=None)
def _make_scatter():
    mesh = plsc.VectorSubcoreMesh(
        core_axis_name="c", subcore_axis_name="s", num_cores=2)

    @functools.partial(
        pl.kernel,
        out_type=jax.ShapeDtypeStruct((NP, GW), F32),
        mesh=mesh,
        compiler_params=pltpu.CompilerParams(needs_layout_passes=False),
        scratch_types=[
            pltpu.VMEM((16,), jnp.int32),
            pltpu.VMEM((CH,), jnp.int32),
            pltpu.VMEM((CH,), jnp.int32),
            pltpu.VMEM((CH,), jnp.int32),
            pltpu.VMEM((CH,), jnp.int32),
            pltpu.VMEM((CH,), jnp.int32),
            pltpu.VMEM((CH, GW), F32),
            pltpu.VMEM((CH, GW), F32),
            pltpu.VMEM((BPT + 8, GW), F32),
            pltpu.SemaphoreType.DMA,
            pltpu.SemaphoreType.DMA,
        ],
    )
    def sk(msg_hbm, lists_hbm, cnts_hbm, out_hbm, cnt_v, pk_v, eid_v, lv_v,
           eid_v2, lv_v2, msg_v, msg_v2, acc, sem, sem2):
        c = lax.axis_index("c")
        s = lax.axis_index("s")
        w = c * 16 + s

        def zero_row(i, carry):
            for k in range(GW // 16):
                acc[i, pl.ds(k * 16, 16)] = jnp.zeros((16,), F32)
            return carry

        lax.fori_loop(0, BPT + 8, zero_row, 0)

        pltpu.sync_copy(cnts_hbm.at[pl.ds(w * 16, 16)], cnt_v)
        cnt = cnt_v[...][0]
        nch = lax.div(cnt + (CH - 1), jnp.int32(CH))

        eid_b = (eid_v, eid_v2)
        lv_b = (lv_v, lv_v2)
        msg_b = (msg_v, msg_v2)
        sem_b = (sem, sem2)

        def load_unpack(jj, b):
            pltpu.sync_copy(lists_hbm.at[pl.ds(w * EPL + jj * CH, CH)], pk_v)

            def unpack(g, carry2):
                pk = pk_v[pl.ds(g * 16, 16)]
                eid_b[b][pl.ds(g * 16, 16)] = lax.shift_right_logical(pk, 9)
                lv_b[b][pl.ds(g * 16, 16)] = lax.bitwise_and(
                    pk, jnp.full((16,), 511, jnp.int32))
                return carry2

            lax.fori_loop(0, CH // 16, unpack, 0)

        def accumulate(b):
            def edge_group(g, carry2):
                lv16 = lv_b[b][pl.ds(g * 16, 16)]
                for e2 in range(16):
                    lv = lv16[e2]
                    row = g * 16 + e2
                    for k in range(GW // 16):
                        plsc.addupdate(acc.at[lv, pl.ds(k * 16, 16)],
                                       msg_b[b][row, pl.ds(k * 16, 16)])
                return carry2

            lax.fori_loop(0, CH // 16, edge_group, 0)

        @pl.when(nch > 0)
        def _():
            load_unpack(jnp.int32(0), 0)
            pltpu.async_copy(msg_hbm.at[eid_b[0]], msg_b[0], sem_b[0])

        def pair(i, carry):
            for b in range(2):
                jj = i * 2 + b
                nb = 1 - b

                @pl.when(jj < nch)
                def _():
                    @pl.when(jj + 1 < nch)
                    def _():
                        load_unpack(jj + 1, nb)
                        pltpu.async_copy(
                            msg_hbm.at[eid_b[nb]], msg_b[nb], sem_b[nb])

                    pltpu.make_async_copy(
                        msg_hbm.at[eid_b[b]], msg_b[b], sem_b[b]).wait()
                    accumulate(b)

            return carry

        lax.fori_loop(0, lax.div(nch + 1, jnp.int32(2)), pair, 0)
        pltpu.sync_copy(acc.at[pl.ds(0, BPT)],
                        out_hbm.at[pl.ds(w * BPT, BPT)])

    return sk


def _dot(a, b, dims, precision=None):
    return lax.dot_general(a, b, (dims, ((), ())),
                           preferred_element_type=F32, precision=precision)


def _prep_call(x_p, batch_2d, mat_flat):
    BN = NP // 4

    def body(xb, bb, mat, out):
        xv = xb[...]
        bcol = bb[...]
        oh = (lax.broadcasted_iota(jnp.int32, (BN, NB), 1)
              == jnp.broadcast_to(bcol, (BN, NB))).astype(F32)
        pos = _dot(oh, mat[...], ((1,), (0,)),
                   precision=lax.Precision.HIGHEST)
        cols = [xv[:, 0:1]]
        for k in range(3):
            col = (xv[:, 1:2] * pos[:, k:k + 1]
                   + xv[:, 2:3] * pos[:, 3 + k:4 + k]
                   + xv[:, 3:4] * pos[:, 6 + k:7 + k])
            cols.append(col)
        cols.append(jnp.zeros((BN, GW - 4), F32))
        out[...] = jnp.concatenate(cols, axis=1)

    return pl.pallas_call(
        body,
        grid=(4,),
        in_specs=[
            pl.BlockSpec((BN, 4), lambda i: (i, 0)),
            pl.BlockSpec((BN, 1), lambda i: (i, 0)),
            pl.BlockSpec((NB, 9), lambda i: (0, 0)),
        ],
        out_specs=pl.BlockSpec((BN, GW), lambda i: (i, 0)),
        out_shape=jax.ShapeDtypeStruct((NP, GW), F32),
    )(x_p, batch_2d, mat_flat)


@functools.lru_cache(maxsize=None)
def _make_conv(ic, oc, BE, IG):
    F = ic * oc
    grid = EP // BE
    ngrp = ic // IG

    def body(hs, hd, hsi, Wa, Wb, out):
        ones = jnp.ones((BE, 1), F32)
        ewc = jnp.concatenate(
            [(hd[...] - hs[...])[:, 1:4], ones], axis=1)
        u1 = _dot(ewc, Wa[...], ((1,), (1,)))
        e1 = u1 * jnp.tanh(u1) + 1.1 * u1
        e1c = jnp.concatenate([e1, ones], axis=1)
        h = hsi[...]
        acc = None
        for grp in range(ngrp):
            lo = grp * IG * oc
            u2 = _dot(e1c, Wb[lo:lo + IG * oc, :], ((1,), (1,)))
            a = u2 * jnp.tanh(u2) + 1.1 * u2
            for i2 in range(IG):
                t = h[:, grp * IG + i2:grp * IG + i2 + 1] * (
                    a[:, i2 * oc:(i2 + 1) * oc])
                acc = t if acc is None else acc + t
        if GW > oc:
            acc = jnp.concatenate(
                [acc, jnp.zeros((BE, GW - oc), F32)], axis=1)
        out[...] = acc

    return pl.pallas_call(
        body,
        grid=(grid,),
        in_specs=[
            pl.BlockSpec((BE, GW), lambda i: (i, 0)),
            pl.BlockSpec((BE, GW), lambda i: (i, 0)),
            pl.BlockSpec((BE, GW), lambda i: (i, 0)),
            pl.BlockSpec((64, 4), lambda i: (0, 0)),
            pl.BlockSpec((F, 65), lambda i: (0, 0)),
        ],
        out_specs=pl.BlockSpec((BE, GW), lambda i: (i, 0)),
        out_shape=jax.ShapeDtypeStruct((EP, GW), F32),
    )


@functools.lru_cache(maxsize=None)
def _make_node(ic, oc):
    BN = NP // 4

    def body(agg, hin, root, bias, W1, b1, W2, b2, out):
        asum = agg[...][:, :oc]
        h = hin[...][:, :ic]
        z = _leaky_silu(asum + _dot(h, root[...], ((1,), (1,)))
                        + bias[...], 0.1)
        val = _leaky_silu(_dot(_leaky_silu(_dot(z, W1[...], ((1,), (1,)))
                                           + b1[...], 0.05),
                               W2[...], ((1,), (1,))) + b2[...], 0.05) + z
        if GW > oc:
            val = jnp.concatenate(
                [val, jnp.zeros((BN, GW - oc), F32)], axis=1)
        out[...] = val

    return pl.pallas_call(
        body,
        grid=(4,),
        in_specs=[
            pl.BlockSpec((BN, GW), lambda i: (i, 0)),
            pl.BlockSpec((BN, GW), lambda i: (i, 0)),
            pl.BlockSpec((oc, ic), lambda i: (0, 0)),
            pl.BlockSpec((1, oc), lambda i: (0, 0)),
            pl.BlockSpec((128, oc), lambda i: (0, 0)),
            pl.BlockSpec((1, 128), lambda i: (0, 0)),
            pl.BlockSpec((oc, 128), lambda i: (0, 0)),
            pl.BlockSpec((1, oc), lambda i: (0, 0)),
        ],
        out_specs=pl.BlockSpec((BN, GW), lambda i: (i, 0)),
        out_shape=jax.ShapeDtypeStruct((NP, GW), F32),
    )


def _readout_call(h3, batch_2d, fcW, fcb):
    BN = NP // 4

    def body(h, bb, W, b, out, g):
        i = pl.program_id(0)

        @pl.when(i == 0)
        def _():
            g[...] = jnp.zeros((NB, 128), F32)

        bcol = bb[...]
        oh = (lax.broadcasted_iota(jnp.int32, (BN, NB), 1)
              == jnp.broadcast_to(bcol, (BN, NB))).astype(F32)
        g[...] += _dot(oh, h[...], ((0,), (0,)),
                       precision=lax.Precision.HIGHEST)
        val = (jnp.sum(g[...] * W[...], axis=1, keepdims=True)
               + jnp.broadcast_to(b[...], (NB, 1)))
        out[...] = -_leaky_silu(val, 0.1)

    return pl.pallas_call(
        body,
        grid=(4,),
        in_specs=[
            pl.BlockSpec((BN, GW), lambda i: (i, 0)),
            pl.BlockSpec((BN, 1), lambda i: (i, 0)),
            pl.BlockSpec((1, 128), lambda i: (0, 0)),
            pl.BlockSpec((1, 1), lambda i: (0, 0)),
        ],
        out_specs=pl.BlockSpec((NB, 1), lambda i: (0, 0)),
        out_shape=jax.ShapeDtypeStruct((NB, 1), F32),
        scratch_shapes=[pltpu.VMEM((NB, 128), F32)],
    )(h3, batch_2d, fcW, fcb)


def kernel(x, edge_index, matrix, batch, params):
    p = params
    i32 = jnp.int32
    src = edge_index[0]
    dst = edge_index[1]
    src_r = jnp.concatenate(
        [src, jnp.zeros((EP - E,), i32)]).reshape(2, 16, NCH, CH)
    dst_p = jnp.concatenate([dst, jnp.full((EP - E,), N, i32)])
    dst_r = dst_p.reshape(2, 16, NCH, CH)
    x_p = jnp.concatenate([x, jnp.zeros((NP - N, 4), F32)], axis=0)
    batch_2d = jnp.concatenate(
        [batch, jnp.full((NP - N,), NB, i32)]).reshape(NP, 1)
    mat_flat = matrix.reshape(NB, 9)

    h0 = _prep_call(x_p, batch_2d, mat_flat)

    gather = _make_gather()
    lists, cnts = _make_bin()(dst_p)
    scatter_b = _make_scatter()

    def scatter(msg, _):
        return scatter_b(msg, lists, cnts)

    hs = gather(h0, src_r)
    hd = gather(h0, dst_r)

    def r2(v):
        return v.reshape(1, -1)

    def wab(Wa, ba, Wb, bb):
        return (jnp.concatenate([Wa, ba[:, None]], axis=1) * 0.5,
                jnp.concatenate([Wb, bb[:, None]], axis=1) * 0.5)

    Wa1, Wb1 = wab(p['c1_Wa'], p['c1_ba'], p['c1_Wb'], p['c1_bb'])
    Wa2, Wb2 = wab(p['c2_Wa'], p['c2_ba'], p['c2_Wb'], p['c2_bb'])
    Wa3, Wb3 = wab(p['c3_Wa'], p['c3_ba'], p['c3_Wb'], p['c3_bb'])

    msg1 = _make_conv(4, 8, 1024, 4)(hs, hd, hs, Wa1, Wb1)
    agg1 = scatter(msg1, dst_r)
    h1 = _make_node(4, 8)(
        agg1, h0, p['c1_root'], r2(p['c1_bias']),
        p['il1_W1'], r2(p['il1_b1']), p['il1_W2'], r2(p['il1_b2']))

    hs8 = gather(h1, src_r)
    msg2 = _make_conv(8, 64, 1024, 8)(hs, hd, hs8, Wa2, Wb2)
    agg2 = scatter(msg2, dst_r)
    h2 = _make_node(8, 64)(
        agg2, h1, p['c2_root'], r2(p['c2_bias']),
        p['il2_W1'], r2(p['il2_b1']), p['il2_W2'], r2(p['il2_b2']))

    hs64 = gather(h2, src_r)
    msg3 = _make_conv(64, 128, 512, 16)(hs, hd, hs64, Wa3, Wb3)
    agg3 = scatter(msg3, dst_r)
    h3 = _make_node(64, 128)(
        agg3, h2, p['c3_root'], r2(p['c3_bias']),
        p['il3_W1'], r2(p['il3_b1']), p['il3_W2'], r2(p['il3_b2']))

    return _readout_call(h3, batch_2d, r2(p['fc1_W']), r2(p['fc1_b']))


# speedup vs baseline: 2.1278x; 1.0132x over previous
import functools

import jax
import jax.numpy as jnp
from jax import lax
from jax.experimental import pallas as pl
from jax.experimental.pallas import tpu as pltpu
from jax.experimental.pallas import tpu_sc as plsc

N = 10000
NP = 10240
E = 50000
EP = 50176
TPW = 1568
NCH = 14
CH = 112
BPT = NP // 32
EPL = EP + 1024
NB = 16
GW = 128
F32 = jnp.float32


def _leaky_silu(v, alpha):
    return v * (0.5 * jnp.tanh(0.5 * v) + (0.5 + alpha))


@functools.lru_cache(maxsize=None)
def _make_gather():
    mesh = plsc.VectorSubcoreMesh(
        core_axis_name="c", subcore_axis_name="s", num_cores=2)

    @functools.partial(
        pl.kernel,
        out_type=jax.ShapeDtypeStruct((EP, GW), F32),
        mesh=mesh,
        scratch_types=[
            pltpu.VMEM((NCH, CH), jnp.int32),
            pltpu.VMEM((CH, GW), F32),
            pltpu.VMEM((CH, GW), F32),
            pltpu.SemaphoreType.DMA,
            pltpu.SemaphoreType.DMA,
        ],
    )
    def gk(table_hbm, idx_hbm, out_hbm, idx_v, buf0, buf1, sem0, sem1):
        c = lax.axis_index("c")
        s = lax.axis_index("s")
        base = (c * 16 + s) * TPW
        pltpu.sync_copy(idx_hbm.at[c, s], idx_v)
        bufs = (buf0, buf1)
        sems = (sem0, sem1)
        cps = [None, None]
        for j in range(NCH):
            k = j % 2
            if cps[k] is not None:
                cps[k].wait()
                pltpu.sync_copy(bufs[k], out_hbm.at[pl.ds(base + (j - 2) * CH, CH)])
            cps[k] = pltpu.async_copy(
                table_hbm.at[idx_v.at[j]], bufs[k], sems[k])
        for j in range(NCH - 2, NCH):
            k = j % 2
            cps[k].wait()
            pltpu.sync_copy(bufs[k], out_hbm.at[pl.ds(base + j * CH, CH)])

    return gk


@functools.lru_cache(maxsize=None)
def _make_bin():
    mesh = plsc.VectorSubcoreMesh(
        core_axis_name="c", subcore_axis_name="s", num_cores=2)

    @functools.partial(
        pl.kernel,
        out_type=(jax.ShapeDtypeStruct((32 * EPL,), jnp.int32),
                  jax.ShapeDtypeStruct((32 * 16,), jnp.int32)),
        mesh=mesh,
        compiler_params=pltpu.CompilerParams(needs_layout_passes=False),
        scratch_types=[
            pltpu.VMEM((TPW,), jnp.int32),
            pltpu.VMEM((EPL,), jnp.int32),
            pltpu.VMEM((16,), jnp.int32),
        ],
    )
    def bk(dst_hbm, lists_hbm, cnts_hbm, idx_v, list_v, cnt_v):
        c = lax.axis_index("c")
        s = lax.axis_index("s")
        w = c * 16 + s
        rbase = w * BPT
        cnt = jnp.int32(0)
        for ch in range(EP // TPW):
            pltpu.sync_copy(dst_hbm.at[pl.ds(ch * TPW, TPW)], idx_v)

            def scan_g(g, cnt):
                v = idx_v[pl.ds(g * 16, 16)]
                lv = v - rbase
                m = (lv >= 0) & (lv < BPT)
                eid = (lax.broadcasted_iota(jnp.int32, (16,), 0)
                       + (ch * TPW + g * 16))
                packed = eid * 512 + jnp.where(m, lv, 0)
                mc = jnp.cumsum(m.astype(jnp.int32))
                plsc.store_scatter(list_v, [cnt + mc - 1], packed, mask=m)
                return cnt + mc[15]

            cnt = lax.fori_loop(0, TPW // 16, scan_g, cnt)
        dummy = jnp.full((16,), BPT, jnp.int32)
        for e in range(CH // 16):
            list_v[pl.ds(cnt + e * 16, 16)] = dummy
        cnt_v[...] = jnp.full((16,), cnt, jnp.int32)
        pltpu.sync_copy(list_v, lists_hbm.at[pl.ds(w * EPL, EPL)])
        pltpu.sync_copy(cnt_v, cnts_hbm.at[pl.ds(w * 16, 16)])

    return bk


@functools.lru_cache(maxsize=None)
def _make_scatter():
    mesh = plsc.VectorSubcoreMesh(
        core_axis_name="c", subcore_axis_name="s", num_cores=2)

    @functools.partial(
        pl.kernel,
        out_type=jax.ShapeDtypeStruct((NP, GW), F32),
        mesh=mesh,
        compiler_params=pltpu.CompilerParams(needs_layout_passes=False),
        scratch_types=[
            pltpu.VMEM((16,), jnp.int32),
            pltpu.VMEM((EPL,), jnp.int32),
            pltpu.VMEM((CH,), jnp.int32),
            pltpu.VMEM((CH,), jnp.int32),
            pltpu.VMEM((CH,), jnp.int32),
            pltpu.VMEM((CH,), jnp.int32),
            pltpu.VMEM((CH, GW), F32),
            pltpu.VMEM((CH, GW), F32),
            pltpu.VMEM((BPT + 8, GW), F32),
            pltpu.SemaphoreType.DMA,
            pltpu.SemaphoreType.DMA,
        ],
    )
    def sk(msg_hbm, lists_hbm, cnts_hbm, out_hbm, cnt_v, pk_v, eid_v, lv_v,
           eid_v2, lv_v2, msg_v, msg_v2, acc, sem, sem2):
        c = lax.axis_index("c")
        s = lax.axis_index("s")
        w = c * 16 + s

        def zero_row(i, carry):
            for k in range(GW // 16):
                acc[i, pl.ds(k * 16, 16)] = jnp.zeros((16,), F32)
            return carry

        lax.fori_loop(0, BPT + 8, zero_row, 0)

        pltpu.sync_copy(cnts_hbm.at[pl.ds(w * 16, 16)], cnt_v)
        cnt = cnt_v[...][0]
        def load_list(j, carry):
            pltpu.sync_copy(lists_hbm.at[pl.ds(w * EPL + j * 1024, 1024)],
                            pk_v.at[pl.ds(j * 1024, 1024)])
            return carry

        lax.fori_loop(0, lax.div(cnt + jnp.int32(CH + 1023), jnp.int32(1024)),
                      load_list, 0)
        nch = lax.div(cnt + (CH - 1), jnp.int32(CH))

        eid_b = (eid_v, eid_v2)
        lv_b = (lv_v, lv_v2)
        msg_b = (msg_v, msg_v2)
        sem_b = (sem, sem2)

        def load_unpack(jj, b):
            def unpack(g, carry2):
                pk = pk_v[pl.ds(jj * CH + g * 16, 16)]
                eid_b[b][pl.ds(g * 16, 16)] = lax.shift_right_logical(pk, 9)
                lv_b[b][pl.ds(g * 16, 16)] = lax.bitwise_and(
                    pk, jnp.full((16,), 511, jnp.int32))
                return carry2

            lax.fori_loop(0, CH // 16, unpack, 0)

        def accumulate(b):
            def edge_group(g, carry2):
                lv16 = lv_b[b][pl.ds(g * 16, 16)]
                for e2 in range(16):
                    lv = lv16[e2]
                    row = g * 16 + e2
                    for k in range(GW // 16):
                        plsc.addupdate(acc.at[lv, pl.ds(k * 16, 16)],
                                       msg_b[b][row, pl.ds(k * 16, 16)])
                return carry2

            lax.fori_loop(0, CH // 16, edge_group, 0)

        @pl.when(nch > 0)
        def _():
            load_unpack(jnp.int32(0), 0)
            pltpu.async_copy(msg_hbm.at[eid_b[0]], msg_b[0], sem_b[0])

        def pair(i, carry):
            for b in range(2):
                jj = i * 2 + b
                nb = 1 - b

                @pl.when(jj < nch)
                def _():
                    @pl.when(jj + 1 < nch)
                    def _():
                        load_unpack(jj + 1, nb)
                        pltpu.async_copy(
                            msg_hbm.at[eid_b[nb]], msg_b[nb], sem_b[nb])

                    pltpu.make_async_copy(
                        msg_hbm.at[eid_b[b]], msg_b[b], sem_b[b]).wait()
                    accumulate(b)

            return carry

        lax.fori_loop(0, lax.div(nch + 1, jnp.int32(2)), pair, 0)
        pltpu.sync_copy(acc.at[pl.ds(0, BPT)],
                        out_hbm.at[pl.ds(w * BPT, BPT)])

    return sk


def _dot(a, b, dims, precision=None):
    return lax.dot_general(a, b, (dims, ((), ())),
                           preferred_element_type=F32, precision=precision)


def _prep_call(x_p, batch_2d, mat_flat):
    BN = NP // 4

    def body(xb, bb, mat, out):
        xv = xb[...]
        bcol = bb[...]
        oh = (lax.broadcasted_iota(jnp.int32, (BN, NB), 1)
              == jnp.broadcast_to(bcol, (BN, NB))).astype(F32)
        pos = _dot(oh, mat[...], ((1,), (0,)),
                   precision=lax.Precision.HIGHEST)
        cols = [xv[:, 0:1]]
        for k in range(3):
            col = (xv[:, 1:2] * pos[:, k:k + 1]
                   + xv[:, 2:3] * pos[:, 3 + k:4 + k]
                   + xv[:, 3:4] * pos[:, 6 + k:7 + k])
            cols.append(col)
        cols.append(jnp.zeros((BN, GW - 4), F32))
        out[...] = jnp.concatenate(cols, axis=1)

    return pl.pallas_call(
        body,
        grid=(4,),
        in_specs=[
            pl.BlockSpec((BN, 4), lambda i: (i, 0)),
            pl.BlockSpec((BN, 1), lambda i: (i, 0)),
            pl.BlockSpec((NB, 9), lambda i: (0, 0)),
        ],
        out_specs=pl.BlockSpec((BN, GW), lambda i: (i, 0)),
        out_shape=jax.ShapeDtypeStruct((NP, GW), F32),
    )(x_p, batch_2d, mat_flat)


@functools.lru_cache(maxsize=None)
def _make_conv(ic, oc, BE, IG):
    F = ic * oc
    grid = EP // BE
    ngrp = ic // IG

    def body(hs, hd, hsi, Wa, Wb, out):
        ones = jnp.ones((BE, 1), F32)
        ewc = jnp.concatenate(
            [(hd[...] - hs[...])[:, 1:4], ones], axis=1)
        u1 = _dot(ewc, Wa[...], ((1,), (1,)))
        e1 = u1 * jnp.tanh(u1) + 1.1 * u1
        e1c = jnp.concatenate([e1, ones], axis=1)
        h = hsi[...]
        acc = None
        for grp in range(ngrp):
            lo = grp * IG * oc
            u2 = _dot(e1c, Wb[lo:lo + IG * oc, :], ((1,), (1,)))
            a = u2 * jnp.tanh(u2) + 1.1 * u2
            for i2 in range(IG):
                t = h[:, grp * IG + i2:grp * IG + i2 + 1] * (
                    a[:, i2 * oc:(i2 + 1) * oc])
                acc = t if acc is None else acc + t
        if GW > oc:
            acc = jnp.concatenate(
                [acc, jnp.zeros((BE, GW - oc), F32)], axis=1)
        out[...] = acc

    return pl.pallas_call(
        body,
        grid=(grid,),
        in_specs=[
            pl.BlockSpec((BE, GW), lambda i: (i, 0)),
            pl.BlockSpec((BE, GW), lambda i: (i, 0)),
            pl.BlockSpec((BE, GW), lambda i: (i, 0)),
            pl.BlockSpec((64, 4), lambda i: (0, 0)),
            pl.BlockSpec((F, 65), lambda i: (0, 0)),
        ],
        out_specs=pl.BlockSpec((BE, GW), lambda i: (i, 0)),
        out_shape=jax.ShapeDtypeStruct((EP, GW), F32),
    )


@functools.lru_cache(maxsize=None)
def _make_node(ic, oc):
    BN = NP // 4

    def body(agg, hin, root, bias, W1, b1, W2, b2, out):
        asum = agg[...][:, :oc]
        h = hin[...][:, :ic]
        z = _leaky_silu(asum + _dot(h, root[...], ((1,), (1,)))
                        + bias[...], 0.1)
        val = _leaky_silu(_dot(_leaky_silu(_dot(z, W1[...], ((1,), (1,)))
                                           + b1[...], 0.05),
                               W2[...], ((1,), (1,))) + b2[...], 0.05) + z
        if GW > oc:
            val = jnp.concatenate(
                [val, jnp.zeros((BN, GW - oc), F32)], axis=1)
        out[...] = val

    return pl.pallas_call(
        body,
        grid=(4,),
        in_specs=[
            pl.BlockSpec((BN, GW), lambda i: (i, 0)),
            pl.BlockSpec((BN, GW), lambda i: (i, 0)),
            pl.BlockSpec((oc, ic), lambda i: (0, 0)),
            pl.BlockSpec((1, oc), lambda i: (0, 0)),
            pl.BlockSpec((128, oc), lambda i: (0, 0)),
            pl.BlockSpec((1, 128), lambda i: (0, 0)),
            pl.BlockSpec((oc, 128), lambda i: (0, 0)),
            pl.BlockSpec((1, oc), lambda i: (0, 0)),
        ],
        out_specs=pl.BlockSpec((BN, GW), lambda i: (i, 0)),
        out_shape=jax.ShapeDtypeStruct((NP, GW), F32),
    )


def _readout_call(h3, batch_2d, fcW, fcb):
    BN = NP // 4

    def body(h, bb, W, b, out, g):
        i = pl.program_id(0)

        @pl.when(i == 0)
        def _():
            g[...] = jnp.zeros((NB, 128), F32)

        bcol = bb[...]
        oh = (lax.broadcasted_iota(jnp.int32, (BN, NB), 1)
              == jnp.broadcast_to(bcol, (BN, NB))).astype(F32)
        g[...] += _dot(oh, h[...], ((0,), (0,)),
                       precision=lax.Precision.HIGHEST)
        val = (jnp.sum(g[...] * W[...], axis=1, keepdims=True)
               + jnp.broadcast_to(b[...], (NB, 1)))
        out[...] = -_leaky_silu(val, 0.1)

    return pl.pallas_call(
        body,
        grid=(4,),
        in_specs=[
            pl.BlockSpec((BN, GW), lambda i: (i, 0)),
            pl.BlockSpec((BN, 1), lambda i: (i, 0)),
            pl.BlockSpec((1, 128), lambda i: (0, 0)),
            pl.BlockSpec((1, 1), lambda i: (0, 0)),
        ],
        out_specs=pl.BlockSpec((NB, 1), lambda i: (0, 0)),
        out_shape=jax.ShapeDtypeStruct((NB, 1), F32),
        scratch_shapes=[pltpu.VMEM((NB, 128), F32)],
    )(h3, batch_2d, fcW, fcb)


def kernel(x, edge_index, matrix, batch, params):
    p = params
    i32 = jnp.int32
    src = edge_index[0]
    dst = edge_index[1]
    src_r = jnp.concatenate(
        [src, jnp.zeros((EP - E,), i32)]).reshape(2, 16, NCH, CH)
    dst_p = jnp.concatenate([dst, jnp.full((EP - E,), N, i32)])
    dst_r = dst_p.reshape(2, 16, NCH, CH)
    x_p = jnp.concatenate([x, jnp.zeros((NP - N, 4), F32)], axis=0)
    batch_2d = jnp.concatenate(
        [batch, jnp.full((NP - N,), NB, i32)]).reshape(NP, 1)
    mat_flat = matrix.reshape(NB, 9)

    h0 = _prep_call(x_p, batch_2d, mat_flat)

    gather = _make_gather()
    lists, cnts = _make_bin()(dst_p)
    scatter_b = _make_scatter()

    def scatter(msg, _):
        return scatter_b(msg, lists, cnts)

    hs = gather(h0, src_r)
    hd = gather(h0, dst_r)

    def r2(v):
        return v.reshape(1, -1)

    def wab(Wa, ba, Wb, bb):
        return (jnp.concatenate([Wa, ba[:, None]], axis=1) * 0.5,
                jnp.concatenate([Wb, bb[:, None]], axis=1) * 0.5)

    Wa1, Wb1 = wab(p['c1_Wa'], p['c1_ba'], p['c1_Wb'], p['c1_bb'])
    Wa2, Wb2 = wab(p['c2_Wa'], p['c2_ba'], p['c2_Wb'], p['c2_bb'])
    Wa3, Wb3 = wab(p['c3_Wa'], p['c3_ba'], p['c3_Wb'], p['c3_bb'])

    msg1 = _make_conv(4, 8, 1024, 4)(hs, hd, hs, Wa1, Wb1)
    agg1 = scatter(msg1, dst_r)
    h1 = _make_node(4, 8)(
        agg1, h0, p['c1_root'], r2(p['c1_bias']),
        p['il1_W1'], r2(p['il1_b1']), p['il1_W2'], r2(p['il1_b2']))

    hs8 = gather(h1, src_r)
    msg2 = _make_conv(8, 64, 1024, 8)(hs, hd, hs8, Wa2, Wb2)
    agg2 = scatter(msg2, dst_r)
    h2 = _make_node(8, 64)(
        agg2, h1, p['c2_root'], r2(p['c2_bias']),
        p['il2_W1'], r2(p['il2_b1']), p['il2_W2'], r2(p['il2_b2']))

    hs64 = gather(h2, src_r)
    msg3 = _make_conv(64, 128, 512, 16)(hs, hd, hs64, Wa3, Wb3)
    agg3 = scatter(msg3, dst_r)
    h3 = _make_node(64, 128)(
        agg3, h2, p['c3_root'], r2(p['c3_bias']),
        p['il3_W1'], r2(p['il3_b1']), p['il3_W2'], r2(p['il3_b2']))

    return _readout_call(h3, batch_2d, r2(p['fc1_W']), r2(p['fc1_b']))


# speedup vs baseline: 2.1369x; 1.0043x over previous
import functools

import jax
import jax.numpy as jnp
from jax import lax
from jax.experimental import pallas as pl
from jax.experimental.pallas import tpu as pltpu
from jax.experimental.pallas import tpu_sc as plsc

N = 10000
NP = 10240
E = 50000
EP = 50176
TPW = 1568
NCH = 14
CH = 112
BPT = NP // 32
EPL = EP + 1024
NB = 16
GW = 128
F32 = jnp.float32


def _leaky_silu(v, alpha):
    return v * (0.5 * jnp.tanh(0.5 * v) + (0.5 + alpha))


@functools.lru_cache(maxsize=None)
def _make_gather():
    mesh = plsc.VectorSubcoreMesh(
        core_axis_name="c", subcore_axis_name="s", num_cores=2)

    @functools.partial(
        pl.kernel,
        out_type=jax.ShapeDtypeStruct((EP, GW), F32),
        mesh=mesh,
        scratch_types=[
            pltpu.VMEM((NCH, CH), jnp.int32),
            pltpu.VMEM((CH, GW), F32),
            pltpu.VMEM((CH, GW), F32),
            pltpu.VMEM((CH, GW), F32),
            pltpu.VMEM((CH, GW), F32),
            pltpu.SemaphoreType.DMA,
            pltpu.SemaphoreType.DMA,
            pltpu.SemaphoreType.DMA,
            pltpu.SemaphoreType.DMA,
        ],
    )
    def gk(table_hbm, idx_hbm, out_hbm, idx_v, buf0, buf1, buf2, buf3,
           sem0, sem1, sem2, sem3):
        c = lax.axis_index("c")
        s = lax.axis_index("s")
        base = (c * 16 + s) * TPW
        pltpu.sync_copy(idx_hbm.at[c, s], idx_v)
        bufs = (buf0, buf1, buf2, buf3)
        sems = (sem0, sem1, sem2, sem3)
        cps = [None, None, None, None]
        for j in range(NCH):
            k = j % 4
            if cps[k] is not None:
                cps[k].wait()
                pltpu.sync_copy(bufs[k],
                                out_hbm.at[pl.ds(base + (j - 4) * CH, CH)])
            cps[k] = pltpu.async_copy(
                table_hbm.at[idx_v.at[j]], bufs[k], sems[k])
        for j in range(NCH - 4, NCH):
            k = j % 4
            cps[k].wait()
            pltpu.sync_copy(bufs[k], out_hbm.at[pl.ds(base + j * CH, CH)])

    return gk


@functools.lru_cache(maxsize=None)
def _make_bin():
    mesh = plsc.VectorSubcoreMesh(
        core_axis_name="c", subcore_axis_name="s", num_cores=2)

    @functools.partial(
        pl.kernel,
        out_type=(jax.ShapeDtypeStruct((32 * EPL,), jnp.int32),
                  jax.ShapeDtypeStruct((32 * 16,), jnp.int32)),
        mesh=mesh,
        compiler_params=pltpu.CompilerParams(needs_layout_passes=False),
        scratch_types=[
            pltpu.VMEM((TPW,), jnp.int32),
            pltpu.VMEM((EPL,), jnp.int32),
            pltpu.VMEM((16,), jnp.int32),
        ],
    )
    def bk(dst_hbm, lists_hbm, cnts_hbm, idx_v, list_v, cnt_v):
        c = lax.axis_index("c")
        s = lax.axis_index("s")
        w = c * 16 + s
        rbase = w * BPT
        cnt = jnp.int32(0)
        for ch in range(EP // TPW):
            pltpu.sync_copy(dst_hbm.at[pl.ds(ch * TPW, TPW)], idx_v)

            def scan_g(g, cnt):
                v = idx_v[pl.ds(g * 16, 16)]
                lv = v - rbase
                m = (lv >= 0) & (lv < BPT)
                eid = (lax.broadcasted_iota(jnp.int32, (16,), 0)
                       + (ch * TPW + g * 16))
                packed = eid * 512 + jnp.where(m, lv, 0)
                mc = jnp.cumsum(m.astype(jnp.int32))
                plsc.store_scatter(list_v, [cnt + mc - 1], packed, mask=m)
                return cnt + mc[15]

            cnt = lax.fori_loop(0, TPW // 16, scan_g, cnt)
        dummy = jnp.full((16,), BPT, jnp.int32)
        for e in range(CH // 16):
            list_v[pl.ds(cnt + e * 16, 16)] = dummy
        cnt_v[...] = jnp.full((16,), cnt, jnp.int32)
        pltpu.sync_copy(list_v, lists_hbm.at[pl.ds(w * EPL, EPL)])
        pltpu.sync_copy(cnt_v, cnts_hbm.at[pl.ds(w * 16, 16)])

    return bk


@functools.lru_cache(maxsize=None)
def _make_scatter():
    mesh = plsc.VectorSubcoreMesh(
        core_axis_name="c", subcore_axis_name="s", num_cores=2)

    @functools.partial(
        pl.kernel,
        out_type=jax.ShapeDtypeStruct((NP, GW), F32),
        mesh=mesh,
        compiler_params=pltpu.CompilerParams(needs_layout_passes=False),
        scratch_types=[
            pltpu.VMEM((16,), jnp.int32),
            pltpu.VMEM((EPL,), jnp.int32),
            pltpu.VMEM((CH,), jnp.int32),
            pltpu.VMEM((CH,), jnp.int32),
            pltpu.VMEM((CH,), jnp.int32),
            pltpu.VMEM((CH,), jnp.int32),
            pltpu.VMEM((CH, GW), F32),
            pltpu.VMEM((CH, GW), F32),
            pltpu.VMEM((BPT + 8, GW), F32),
            pltpu.SemaphoreType.DMA,
            pltpu.SemaphoreType.DMA,
        ],
    )
    def sk(msg_hbm, lists_hbm, cnts_hbm, out_hbm, cnt_v, pk_v, eid_v, lv_v,
           eid_v2, lv_v2, msg_v, msg_v2, acc, sem, sem2):
        c = lax.axis_index("c")
        s = lax.axis_index("s")
        w = c * 16 + s

        def zero_row(i, carry):
            for k in range(GW // 16):
                acc[i, pl.ds(k * 16, 16)] = jnp.zeros((16,), F32)
            return carry

        lax.fori_loop(0, BPT + 8, zero_row, 0)

        pltpu.sync_copy(cnts_hbm.at[pl.ds(w * 16, 16)], cnt_v)
        cnt = cnt_v[...][0]
        def load_list(j, carry):
            pltpu.sync_copy(lists_hbm.at[pl.ds(w * EPL + j * 1024, 1024)],
                            pk_v.at[pl.ds(j * 1024, 1024)])
            return carry

        lax.fori_loop(0, lax.div(cnt + jnp.int32(CH + 1023), jnp.int32(1024)),
                      load_list, 0)
        nch = lax.div(cnt + (CH - 1), jnp.int32(CH))

        eid_b = (eid_v, eid_v2)
        lv_b = (lv_v, lv_v2)
        msg_b = (msg_v, msg_v2)
        sem_b = (sem, sem2)

        def load_unpack(jj, b):
            def unpack(g, carry2):
                pk = pk_v[pl.ds(jj * CH + g * 16, 16)]
                eid_b[b][pl.ds(g * 16, 16)] = lax.shift_right_logical(pk, 9)
                lv_b[b][pl.ds(g * 16, 16)] = lax.bitwise_and(
                    pk, jnp.full((16,), 511, jnp.int32))
                return carry2

            lax.fori_loop(0, CH // 16, unpack, 0)

        def accumulate(b):
            def edge_group(g, carry2):
                lv16 = lv_b[b][pl.ds(g * 16, 16)]
                for e2 in range(16):
                    lv = lv16[e2]
                    row = g * 16 + e2
                    for k in range(GW // 16):
                        plsc.addupdate(acc.at[lv, pl.ds(k * 16, 16)],
                                       msg_b[b][row, pl.ds(k * 16, 16)])
                return carry2

            lax.fori_loop(0, CH // 16, edge_group, 0)

        @pl.when(nch > 0)
        def _():
            load_unpack(jnp.int32(0), 0)
            pltpu.async_copy(msg_hbm.at[eid_b[0]], msg_b[0], sem_b[0])

        def pair(i, carry):
            for b in range(2):
                jj = i * 2 + b
                nb = 1 - b

                @pl.when(jj < nch)
                def _():
                    @pl.when(jj + 1 < nch)
                    def _():
                        load_unpack(jj + 1, nb)
                        pltpu.async_copy(
                            msg_hbm.at[eid_b[nb]], msg_b[nb], sem_b[nb])

                    pltpu.make_async_copy(
                        msg_hbm.at[eid_b[b]], msg_b[b], sem_b[b]).wait()
                    accumulate(b)

            return carry

        lax.fori_loop(0, lax.div(nch + 1, jnp.int32(2)), pair, 0)
        pltpu.sync_copy(acc.at[pl.ds(0, BPT)],
                        out_hbm.at[pl.ds(w * BPT, BPT)])

    return sk


def _dot(a, b, dims, precision=None):
    return lax.dot_general(a, b, (dims, ((), ())),
                           preferred_element_type=F32, precision=precision)


def _prep_call(x_p, batch_2d, mat_flat):
    BN = NP // 4

    def body(xb, bb, mat, out):
        xv = xb[...]
        bcol = bb[...]
        oh = (lax.broadcasted_iota(jnp.int32, (BN, NB), 1)
              == jnp.broadcast_to(bcol, (BN, NB))).astype(F32)
        pos = _dot(oh, mat[...], ((1,), (0,)),
                   precision=lax.Precision.HIGHEST)
        cols = [xv[:, 0:1]]
        for k in range(3):
            col = (xv[:, 1:2] * pos[:, k:k + 1]
                   + xv[:, 2:3] * pos[:, 3 + k:4 + k]
                   + xv[:, 3:4] * pos[:, 6 + k:7 + k])
            cols.append(col)
        cols.append(jnp.zeros((BN, GW - 4), F32))
        out[...] = jnp.concatenate(cols, axis=1)

    return pl.pallas_call(
        body,
        grid=(4,),
        in_specs=[
            pl.BlockSpec((BN, 4), lambda i: (i, 0)),
            pl.BlockSpec((BN, 1), lambda i: (i, 0)),
            pl.BlockSpec((NB, 9), lambda i: (0, 0)),
        ],
        out_specs=pl.BlockSpec((BN, GW), lambda i: (i, 0)),
        out_shape=jax.ShapeDtypeStruct((NP, GW), F32),
    )(x_p, batch_2d, mat_flat)


@functools.lru_cache(maxsize=None)
def _make_conv(ic, oc, BE, IG):
    F = ic * oc
    grid = EP // BE
    ngrp = ic // IG

    def body(hs, hd, hsi, Wa, Wb, out):
        ones = jnp.ones((BE, 1), F32)
        ewc = jnp.concatenate(
            [(hd[...] - hs[...])[:, 1:4], ones], axis=1)
        u1 = _dot(ewc, Wa[...], ((1,), (1,)))
        e1 = u1 * jnp.tanh(u1) + 1.1 * u1
        e1c = jnp.concatenate([e1, ones], axis=1)
        h = hsi[...]
        acc = None
        for grp in range(ngrp):
            lo = grp * IG * oc
            u2 = _dot(e1c, Wb[lo:lo + IG * oc, :], ((1,), (1,)))
            a = u2 * jnp.tanh(u2) + 1.1 * u2
            for i2 in range(IG):
                t = h[:, grp * IG + i2:grp * IG + i2 + 1] * (
                    a[:, i2 * oc:(i2 + 1) * oc])
                acc = t if acc is None else acc + t
        if GW > oc:
            acc = jnp.concatenate(
                [acc, jnp.zeros((BE, GW - oc), F32)], axis=1)
        out[...] = acc

    return pl.pallas_call(
        body,
        grid=(grid,),
        in_specs=[
            pl.BlockSpec((BE, GW), lambda i: (i, 0)),
            pl.BlockSpec((BE, GW), lambda i: (i, 0)),
            pl.BlockSpec((BE, GW), lambda i: (i, 0)),
            pl.BlockSpec((64, 4), lambda i: (0, 0)),
            pl.BlockSpec((F, 65), lambda i: (0, 0)),
        ],
        out_specs=pl.BlockSpec((BE, GW), lambda i: (i, 0)),
        out_shape=jax.ShapeDtypeStruct((EP, GW), F32),
    )


@functools.lru_cache(maxsize=None)
def _make_node(ic, oc):
    BN = NP // 4

    def body(agg, hin, root, bias, W1, b1, W2, b2, out):
        asum = agg[...][:, :oc]
        h = hin[...][:, :ic]
        z = _leaky_silu(asum + _dot(h, root[...], ((1,), (1,)))
                        + bias[...], 0.1)
        val = _leaky_silu(_dot(_leaky_silu(_dot(z, W1[...], ((1,), (1,)))
                                           + b1[...], 0.05),
                               W2[...], ((1,), (1,))) + b2[...], 0.05) + z
        if GW > oc:
            val = jnp.concatenate(
                [val, jnp.zeros((BN, GW - oc), F32)], axis=1)
        out[...] = val

    return pl.pallas_call(
        body,
        grid=(4,),
        in_specs=[
            pl.BlockSpec((BN, GW), lambda i: (i, 0)),
            pl.BlockSpec((BN, GW), lambda i: (i, 0)),
            pl.BlockSpec((oc, ic), lambda i: (0, 0)),
            pl.BlockSpec((1, oc), lambda i: (0, 0)),
            pl.BlockSpec((128, oc), lambda i: (0, 0)),
            pl.BlockSpec((1, 128), lambda i: (0, 0)),
            pl.BlockSpec((oc, 128), lambda i: (0, 0)),
            pl.BlockSpec((1, oc), lambda i: (0, 0)),
        ],
        out_specs=pl.BlockSpec((BN, GW), lambda i: (i, 0)),
        out_shape=jax.ShapeDtypeStruct((NP, GW), F32),
    )


def _readout_call(h3, batch_2d, fcW, fcb):
    BN = NP // 4

    def body(h, bb, W, b, out, g):
        i = pl.program_id(0)

        @pl.when(i == 0)
        def _():
            g[...] = jnp.zeros((NB, 128), F32)

        bcol = bb[...]
        oh = (lax.broadcasted_iota(jnp.int32, (BN, NB), 1)
              == jnp.broadcast_to(bcol, (BN, NB))).astype(F32)
        g[...] += _dot(oh, h[...], ((0,), (0,)),
                       precision=lax.Precision.HIGHEST)
        val = (jnp.sum(g[...] * W[...], axis=1, keepdims=True)
               + jnp.broadcast_to(b[...], (NB, 1)))
        out[...] = -_leaky_silu(val, 0.1)

    return pl.pallas_call(
        body,
        grid=(4,),
        in_specs=[
            pl.BlockSpec((BN, GW), lambda i: (i, 0)),
            pl.BlockSpec((BN, 1), lambda i: (i, 0)),
            pl.BlockSpec((1, 128), lambda i: (0, 0)),
            pl.BlockSpec((1, 1), lambda i: (0, 0)),
        ],
        out_specs=pl.BlockSpec((NB, 1), lambda i: (0, 0)),
        out_shape=jax.ShapeDtypeStruct((NB, 1), F32),
        scratch_shapes=[pltpu.VMEM((NB, 128), F32)],
    )(h3, batch_2d, fcW, fcb)


def kernel(x, edge_index, matrix, batch, params):
    p = params
    i32 = jnp.int32
    src = edge_index[0]
    dst = edge_index[1]
    src_r = jnp.concatenate(
        [src, jnp.zeros((EP - E,), i32)]).reshape(2, 16, NCH, CH)
    dst_p = jnp.concatenate([dst, jnp.full((EP - E,), N, i32)])
    dst_r = dst_p.reshape(2, 16, NCH, CH)
    x_p = jnp.concatenate([x, jnp.zeros((NP - N, 4), F32)], axis=0)
    batch_2d = jnp.concatenate(
        [batch, jnp.full((NP - N,), NB, i32)]).reshape(NP, 1)
    mat_flat = matrix.reshape(NB, 9)

    h0 = _prep_call(x_p, batch_2d, mat_flat)

    gather = _make_gather()
    lists, cnts = _make_bin()(dst_p)
    scatter_b = _make_scatter()

    def scatter(msg, _):
        return scatter_b(msg, lists, cnts)

    hs = gather(h0, src_r)
    hd = gather(h0, dst_r)

    def r2(v):
        return v.reshape(1, -1)

    def wab(Wa, ba, Wb, bb):
        return (jnp.concatenate([Wa, ba[:, None]], axis=1) * 0.5,
                jnp.concatenate([Wb, bb[:, None]], axis=1) * 0.5)

    Wa1, Wb1 = wab(p['c1_Wa'], p['c1_ba'], p['c1_Wb'], p['c1_bb'])
    Wa2, Wb2 = wab(p['c2_Wa'], p['c2_ba'], p['c2_Wb'], p['c2_bb'])
    Wa3, Wb3 = wab(p['c3_Wa'], p['c3_ba'], p['c3_Wb'], p['c3_bb'])

    msg1 = _make_conv(4, 8, 1024, 4)(hs, hd, hs, Wa1, Wb1)
    agg1 = scatter(msg1, dst_r)
    h1 = _make_node(4, 8)(
        agg1, h0, p['c1_root'], r2(p['c1_bias']),
        p['il1_W1'], r2(p['il1_b1']), p['il1_W2'], r2(p['il1_b2']))

    hs8 = gather(h1, src_r)
    msg2 = _make_conv(8, 64, 1024, 8)(hs, hd, hs8, Wa2, Wb2)
    agg2 = scatter(msg2, dst_r)
    h2 = _make_node(8, 64)(
        agg2, h1, p['c2_root'], r2(p['c2_bias']),
        p['il2_W1'], r2(p['il2_b1']), p['il2_W2'], r2(p['il2_b2']))

    hs64 = gather(h2, src_r)
    msg3 = _make_conv(64, 128, 512, 16)(hs, hd, hs64, Wa3, Wb3)
    agg3 = scatter(msg3, dst_r)
    h3 = _make_node(64, 128)(
        agg3, h2, p['c3_root'], r2(p['c3_bias']),
        p['il3_W1'], r2(p['il3_b1']), p['il3_W2'], r2(p['il3_b2']))

    return _readout_call(h3, batch_2d, r2(p['fc1_W']), r2(p['fc1_b']))
